# baseline (speedup 1.0000x reference)
import functools

import jax
import jax.numpy as jnp
import numpy as np
from jax import lax
from jax.experimental import pallas as pl
from jax.experimental.pallas import tpu as pltpu
from jax.experimental.pallas import tpu_sc as plsc

N = 10000
E = 320000
D = 128
DE = 16
EPS = 1e-05

NC = 2
NS = 16
NW = NC * NS
EPW = E // NW
K = 128
NFULL = EPW // K
TE = EPW - NFULL * K
RO = 624
ZR = 48
TAIL = N - NS * RO

_MESH = plsc.VectorSubcoreMesh(core_axis_name="c", subcore_axis_name="s")
_SC_PARAMS = pltpu.CompilerParams(use_tc_tiling_on_sc=False)


def _zero_fill(zbuf, width):
    zv = jnp.zeros((16,), jnp.float32)

    @pl.loop(0, ZR)
    def _(i):
        @pl.loop(0, width // 16)
        def _(j):
            zbuf[i, pl.ds(j * 16, 16)] = zv


def _zero_shared(sid, zbuf, sh):
    rbase = sid * RO
    for r in range(RO // ZR):
        pltpu.sync_copy(zbuf, sh.at[pl.ds(rbase + r * ZR, ZR)])

    @pl.when(sid == NS - 1)
    def _():
        pltpu.sync_copy(zbuf.at[pl.ds(0, TAIL)], sh.at[pl.ds(NS * RO, TAIL)])


def _write_out(sid, cid, sh, out):
    rbase = sid * RO
    pltpu.sync_copy(sh.at[pl.ds(rbase, RO)], out.at[cid, pl.ds(rbase, RO)])

    @pl.when(sid == NS - 1)
    def _():
        pltpu.sync_copy(sh.at[pl.ds(NS * RO, TAIL)], out.at[cid, pl.ds(NS * RO, TAIL)])


def _make_spmm():
    scratch = [
        [pltpu.VMEM((K,), jnp.int32)] * 2,
        [pltpu.VMEM((K,), jnp.int32)] * 2,
        [pltpu.VMEM((K, D), jnp.float32)] * 2,
        pltpu.VMEM((TE,), jnp.int32),
        pltpu.VMEM((TE,), jnp.int32),
        pltpu.VMEM((TE, D), jnp.float32),
        pltpu.VMEM((ZR, D), jnp.float32),
        pltpu.VMEM_SHARED((N, D), jnp.float32),
        [pltpu.SemaphoreType.DMA] * 2,
        [pltpu.SemaphoreType.DMA] * 2,
        pltpu.SemaphoreType.DMA,
    ]

    def body(h_hbm, ei_hbm, agg_out, dstv, srcv, rows, dstv_t, srcv_t, rows_t,
             zbuf, agg_sh, sem_i, sem_g, sem_t):
        cid = lax.axis_index("c")
        sid = lax.axis_index("s")
        _zero_fill(zbuf, D)
        _zero_shared(sid, zbuf, agg_sh)
        plsc.subcore_barrier()

        ebase = (sid * NC + cid) * EPW

        def start_inputs(c, b):
            off = ebase + c * K
            pltpu.async_copy(ei_hbm.at[pl.ds(off, K)], dstv[b], sem_i[b])
            pltpu.async_copy(ei_hbm.at[pl.ds(E + off, K)], srcv[b], sem_i[b])

        def wait_inputs(b):
            pltpu.make_async_copy(ei_hbm.at[pl.ds(0, K)], dstv[b], sem_i[b]).wait()
            pltpu.make_async_copy(ei_hbm.at[pl.ds(0, K)], srcv[b], sem_i[b]).wait()

        def start_gather(b):
            pltpu.async_copy(h_hbm.at[srcv[b]], rows[b], sem_g[b])

        def wait_gather(b):
            pltpu.make_async_copy(h_hbm.at[srcv[b]], rows[b], sem_g[b]).wait()

        def scatter(b):
            pltpu.sync_copy(rows[b], agg_sh.at[dstv[b]], add=True)

        start_inputs(0, 0)
        start_inputs(1, 1)
        wait_inputs(0)
        start_gather(0)

        @pl.loop(0, NFULL - 2, step=2)
        def _(j):
            for b in (0, 1):
                c = j + b
                wait_inputs(1 - b)
                start_gather(1 - b)
                wait_gather(b)
                scatter(b)
                start_inputs(c + 2, b)

        wait_inputs(1)
        start_gather(1)
        wait_gather(0)
        scatter(0)
        wait_gather(1)
        scatter(1)

        toff = ebase + NFULL * K
        pltpu.sync_copy(ei_hbm.at[pl.ds(toff, TE)], dstv_t)
        pltpu.sync_copy(ei_hbm.at[pl.ds(E + toff, TE)], srcv_t)
        pltpu.async_copy(h_hbm.at[srcv_t], rows_t, sem_t).wait()
        pltpu.sync_copy(rows_t, agg_sh.at[dstv_t], add=True)

        plsc.subcore_barrier()
        _write_out(sid, cid, agg_sh, agg_out)

    return functools.partial(
        pl.kernel, mesh=_MESH,
        out_type=(jax.ShapeDtypeStruct((NC, N, D), jnp.float32),),
        scratch_types=scratch, compiler_params=_SC_PARAMS)(body)


def _make_sdeg():
    scratch = [
        [pltpu.VMEM((K,), jnp.int32)] * 2,
        [pltpu.VMEM((K, DE), jnp.float32)] * 2,
        pltpu.VMEM((K, DE), jnp.float32),
        pltpu.VMEM((TE,), jnp.int32),
        pltpu.VMEM((TE, DE), jnp.float32),
        pltpu.VMEM((TE, DE), jnp.float32),
        pltpu.VMEM((ZR, DE), jnp.float32),
        pltpu.VMEM_SHARED((N, DE), jnp.float32),
        pltpu.VMEM_SHARED((N, DE), jnp.float32),
        [pltpu.SemaphoreType.DMA] * 2,
    ]

    def body(ei_hbm, ea_hbm, s_out, deg_out, dstv, eav, ones, dstv_t, eav_t,
             ones_t, zbuf16, s_sh, deg_sh, sem_i):
        cid = lax.axis_index("c")
        sid = lax.axis_index("s")
        ov = jnp.ones((16,), jnp.float32)

        @pl.loop(0, K)
        def _(i):
            ones[i, pl.ds(0, 16)] = ov

        @pl.loop(0, TE)
        def _(i):
            ones_t[i, pl.ds(0, 16)] = ov

        _zero_fill(zbuf16, DE)
        _zero_shared(sid, zbuf16, s_sh)
        _zero_shared(sid, zbuf16, deg_sh)
        plsc.subcore_barrier()

        ebase = (sid * NC + cid) * EPW

        def start_inputs(c, b):
            off = ebase + c * K
            pltpu.async_copy(ei_hbm.at[pl.ds(off, K)], dstv[b], sem_i[b])
            pltpu.async_copy(ea_hbm.at[pl.ds(off, K)], eav[b], sem_i[b])

        def wait_inputs(b):
            pltpu.make_async_copy(ei_hbm.at[pl.ds(0, K)], dstv[b], sem_i[b]).wait()
            pltpu.make_async_copy(ea_hbm.at[pl.ds(0, K)], eav[b], sem_i[b]).wait()

        def scatter(b):
            pltpu.sync_copy(eav[b], s_sh.at[dstv[b]], add=True)
            pltpu.sync_copy(ones, deg_sh.at[dstv[b]], add=True)

        start_inputs(0, 0)
        start_inputs(1, 1)

        @pl.loop(0, NFULL - 2, step=2)
        def _(j):
            for b in (0, 1):
                c = j + b
                wait_inputs(b)
                scatter(b)
                start_inputs(c + 2, b)

        wait_inputs(0)
        scatter(0)
        wait_inputs(1)
        scatter(1)

        toff = ebase + NFULL * K
        pltpu.sync_copy(ei_hbm.at[pl.ds(toff, TE)], dstv_t)
        pltpu.sync_copy(ea_hbm.at[pl.ds(toff, TE)], eav_t)
        pltpu.sync_copy(eav_t, s_sh.at[dstv_t], add=True)
        pltpu.sync_copy(ones_t, deg_sh.at[dstv_t], add=True)

        plsc.subcore_barrier()
        _write_out(sid, cid, s_sh, s_out)
        _write_out(sid, cid, deg_sh, deg_out)

    return functools.partial(
        pl.kernel, mesh=_MESH,
        out_type=(jax.ShapeDtypeStruct((NC, N, DE), jnp.float32),
                  jax.ShapeDtypeStruct((NC, N, DE), jnp.float32)),
        scratch_types=scratch, compiler_params=_SC_PARAMS)(body)


_spmm = _make_spmm()
_sdeg = _make_sdeg()


def _make_mlp(final_relu: bool, with_sd_inputs: bool):
    R = 2000

    def body(agg_ref, h_ref, s_ref, d_ref, a_ref, b_ref, v_ref, u_ref,
             w2_ref, b2_ref, o_ref):
        z = agg_ref[0] + agg_ref[1] + h_ref[...]
        sarr = s_ref[0] + s_ref[1]
        darr = d_ref[0] + d_ref[1]
        dcol = darr[:, :1]
        pre = (jnp.dot(z, a_ref[...], preferred_element_type=jnp.float32)
               + jnp.dot(sarr, b_ref[...], preferred_element_type=jnp.float32)
               + dcol * v_ref[...] + u_ref[...])
        t = jnp.maximum(pre, 0.0)
        out = jnp.dot(t, w2_ref[...], preferred_element_type=jnp.float32) + b2_ref[...]
        if final_relu:
            out = jnp.maximum(out, 0.0)
        o_ref[...] = out

    grid = (N // R,)
    in_specs = [
        pl.BlockSpec((NC, R, D), lambda i: (0, i, 0)),
        pl.BlockSpec((R, D), lambda i: (i, 0)),
        pl.BlockSpec((NC, R, DE), lambda i: (0, i, 0)),
        pl.BlockSpec((NC, R, DE), lambda i: (0, i, 0)),
        pl.BlockSpec((D, 2 * D), lambda i: (0, 0)),
        pl.BlockSpec((DE, 2 * D), lambda i: (0, 0)),
        pl.BlockSpec((1, 2 * D), lambda i: (0, 0)),
        pl.BlockSpec((1, 2 * D), lambda i: (0, 0)),
        pl.BlockSpec((2 * D, D), lambda i: (0, 0)),
        pl.BlockSpec((1, D), lambda i: (0, 0)),
    ]
    return pl.pallas_call(
        body,
        grid=grid,
        in_specs=in_specs,
        out_specs=pl.BlockSpec((R, D), lambda i: (i, 0)),
        out_shape=jax.ShapeDtypeStruct((N, D), jnp.float32),
    )


_mlp0 = _make_mlp(final_relu=True, with_sd_inputs=True)
_mlp1 = _make_mlp(final_relu=False, with_sd_inputs=True)

_SCALE = 1.0 / np.sqrt(1.0 + EPS)


def kernel(x, edge_index, edge_attr, self_loop_index, self_loop_type,
           W_enc0, b_enc0, W1_0, b1_0, gamma0, beta0, W2_0, b2_0,
           W_enc1, b_enc1, W1_1, b1_1, gamma1, beta1, W2_1, b2_1):
    sl_row = ((jnp.arange(DE) == self_loop_index).astype(jnp.float32)
              * jnp.asarray(self_loop_type, jnp.float32))

    def fold(W1, b1, gamma, beta):
        g = gamma * _SCALE
        return W1 * g[None, :], b1 * g + beta

    W1f0, b1f0 = fold(W1_0, b1_0, gamma0, beta0)
    A0 = W1f0
    B0 = W_enc0 @ W1f0
    v0 = (b_enc0 @ W1f0)[None, :]
    u0 = ((sl_row @ W_enc0 + b_enc0) @ W1f0 + b1f0)[None, :]

    W1f1, b1f1 = fold(W1_1, b1_1, gamma1, beta1)
    A1 = W1f1[:D]
    Wb = W1f1[D:]
    B1 = W_enc1 @ Wb
    v1 = (b_enc1 @ Wb)[None, :]
    u1 = ((sl_row @ W_enc1 + b_enc1) @ Wb + b1f1)[None, :]

    ei_lin = edge_index.reshape(2 * E)
    (aggx,) = _spmm(x, ei_lin)
    ei_lin2, _ = lax.optimization_barrier((ei_lin, aggx))
    S, deg = _sdeg(ei_lin2, edge_attr)
    h0 = _mlp0(aggx, x, S, deg, A0, B0, v0, u0, W2_0, b2_0[None, :])
    (aggh,) = _spmm(h0, ei_lin)
    h1 = _mlp1(aggh, h0, S, deg, A1, B1, v1, u1, W2_1, b2_1[None, :])
    return h1


# speedup vs baseline: 1.1598x; 1.1598x over previous
import functools

import jax
import jax.numpy as jnp
import numpy as np
from jax import lax
from jax.experimental import pallas as pl
from jax.experimental.pallas import tpu as pltpu
from jax.experimental.pallas import tpu_sc as plsc

N = 10000
E = 320000
D = 128
DE = 16
EPS = 1e-05

NC = 2
NS = 16
NW = NC * NS
EPW = E // NW
K = 128
NFULL = EPW // K
TE = EPW - NFULL * K
RO = 624
ZR = 48
TAIL = N - NS * RO

_MESH = plsc.VectorSubcoreMesh(core_axis_name="c", subcore_axis_name="s")
_SC_PARAMS = pltpu.CompilerParams(use_tc_tiling_on_sc=False)


def _zero_fill(zbuf, width):
    zv = jnp.zeros((16,), jnp.float32)

    @pl.loop(0, ZR)
    def _(i):
        @pl.loop(0, width // 16)
        def _(j):
            zbuf[i, pl.ds(j * 16, 16)] = zv


def _zero_shared(sid, zbuf, sh):
    rbase = sid * RO
    for r in range(RO // ZR):
        pltpu.sync_copy(zbuf, sh.at[pl.ds(rbase + r * ZR, ZR)])

    @pl.when(sid == NS - 1)
    def _():
        pltpu.sync_copy(zbuf.at[pl.ds(0, TAIL)], sh.at[pl.ds(NS * RO, TAIL)])


def _write_out(sid, cid, sh, out):
    rbase = sid * RO
    pltpu.sync_copy(sh.at[pl.ds(rbase, RO)], out.at[cid, pl.ds(rbase, RO)])

    @pl.when(sid == NS - 1)
    def _():
        pltpu.sync_copy(sh.at[pl.ds(NS * RO, TAIL)], out.at[cid, pl.ds(NS * RO, TAIL)])


def _make_spmm():
    scratch = [
        [pltpu.VMEM((K,), jnp.int32)] * 2,
        [pltpu.VMEM((K,), jnp.int32)] * 2,
        [pltpu.VMEM((K, D), jnp.float32)] * 2,
        pltpu.VMEM((TE,), jnp.int32),
        pltpu.VMEM((TE,), jnp.int32),
        pltpu.VMEM((TE, D), jnp.float32),
        pltpu.VMEM((ZR, D), jnp.float32),
        pltpu.VMEM_SHARED((N, D), jnp.float32),
        [pltpu.SemaphoreType.DMA] * 2,
        [pltpu.SemaphoreType.DMA] * 2,
        pltpu.SemaphoreType.DMA,
    ]

    def body(h_hbm, ei_hbm, agg_out, dstv, srcv, rows, dstv_t, srcv_t, rows_t,
             zbuf, agg_sh, sem_i, sem_g, sem_t):
        cid = lax.axis_index("c")
        sid = lax.axis_index("s")
        _zero_fill(zbuf, D)
        _zero_shared(sid, zbuf, agg_sh)
        plsc.subcore_barrier()

        ebase = (sid * NC + cid) * EPW

        def start_inputs(c, b):
            off = ebase + c * K
            pltpu.async_copy(ei_hbm.at[pl.ds(off, K)], dstv[b], sem_i[b])
            pltpu.async_copy(ei_hbm.at[pl.ds(E + off, K)], srcv[b], sem_i[b])

        def wait_inputs(b):
            pltpu.make_async_copy(ei_hbm.at[pl.ds(0, K)], dstv[b], sem_i[b]).wait()
            pltpu.make_async_copy(ei_hbm.at[pl.ds(0, K)], srcv[b], sem_i[b]).wait()

        def start_gather(b):
            pltpu.async_copy(h_hbm.at[srcv[b]], rows[b], sem_g[b])

        def wait_gather(b):
            pltpu.make_async_copy(h_hbm.at[srcv[b]], rows[b], sem_g[b]).wait()

        def scatter(b):
            pltpu.sync_copy(rows[b], agg_sh.at[dstv[b]], add=True)

        start_inputs(0, 0)
        start_inputs(1, 1)
        wait_inputs(0)
        start_gather(0)

        @pl.loop(0, NFULL - 2, step=2)
        def _(j):
            for b in (0, 1):
                c = j + b
                wait_inputs(1 - b)
                start_gather(1 - b)
                wait_gather(b)
                scatter(b)
                start_inputs(c + 2, b)

        wait_inputs(1)
        start_gather(1)
        wait_gather(0)
        scatter(0)
        wait_gather(1)
        scatter(1)

        toff = ebase + NFULL * K
        pltpu.sync_copy(ei_hbm.at[pl.ds(toff, TE)], dstv_t)
        pltpu.sync_copy(ei_hbm.at[pl.ds(E + toff, TE)], srcv_t)
        pltpu.async_copy(h_hbm.at[srcv_t], rows_t, sem_t).wait()
        pltpu.sync_copy(rows_t, agg_sh.at[dstv_t], add=True)

        plsc.subcore_barrier()
        _write_out(sid, cid, agg_sh, agg_out)

    return functools.partial(
        pl.kernel, mesh=_MESH,
        out_type=(jax.ShapeDtypeStruct((NC, N, D), jnp.float32),),
        scratch_types=scratch, compiler_params=_SC_PARAMS)(body)


def _make_sdeg():
    scratch = [
        [pltpu.VMEM((K,), jnp.int32)] * 2,
        [pltpu.VMEM((K, DE), jnp.float32)] * 2,
        pltpu.VMEM((K, DE), jnp.float32),
        pltpu.VMEM((TE,), jnp.int32),
        pltpu.VMEM((TE, DE), jnp.float32),
        pltpu.VMEM((TE, DE), jnp.float32),
        pltpu.VMEM((ZR, DE), jnp.float32),
        pltpu.VMEM_SHARED((N, DE), jnp.float32),
        pltpu.VMEM_SHARED((N, DE), jnp.float32),
        [pltpu.SemaphoreType.DMA] * 2,
    ]

    def body(ei_hbm, ea_hbm, s_out, deg_out, dstv, eav, ones, dstv_t, eav_t,
             ones_t, zbuf16, s_sh, deg_sh, sem_i):
        cid = lax.axis_index("c")
        sid = lax.axis_index("s")
        ov = jnp.ones((16,), jnp.float32)

        @pl.loop(0, K)
        def _(i):
            ones[i, pl.ds(0, 16)] = ov

        @pl.loop(0, TE)
        def _(i):
            ones_t[i, pl.ds(0, 16)] = ov

        _zero_fill(zbuf16, DE)
        _zero_shared(sid, zbuf16, s_sh)
        _zero_shared(sid, zbuf16, deg_sh)
        plsc.subcore_barrier()

        ebase = (sid * NC + cid) * EPW

        def start_inputs(c, b):
            off = ebase + c * K
            pltpu.async_copy(ei_hbm.at[pl.ds(off, K)], dstv[b], sem_i[b])
            pltpu.async_copy(ea_hbm.at[pl.ds(off, K)], eav[b], sem_i[b])

        def wait_inputs(b):
            pltpu.make_async_copy(ei_hbm.at[pl.ds(0, K)], dstv[b], sem_i[b]).wait()
            pltpu.make_async_copy(ea_hbm.at[pl.ds(0, K)], eav[b], sem_i[b]).wait()

        def scatter(b):
            pltpu.sync_copy(eav[b], s_sh.at[dstv[b]], add=True)
            pltpu.sync_copy(ones, deg_sh.at[dstv[b]], add=True)

        start_inputs(0, 0)
        start_inputs(1, 1)

        @pl.loop(0, NFULL - 2, step=2)
        def _(j):
            for b in (0, 1):
                c = j + b
                wait_inputs(b)
                scatter(b)
                start_inputs(c + 2, b)

        wait_inputs(0)
        scatter(0)
        wait_inputs(1)
        scatter(1)

        toff = ebase + NFULL * K
        pltpu.sync_copy(ei_hbm.at[pl.ds(toff, TE)], dstv_t)
        pltpu.sync_copy(ea_hbm.at[pl.ds(toff, TE)], eav_t)
        pltpu.sync_copy(eav_t, s_sh.at[dstv_t], add=True)
        pltpu.sync_copy(ones_t, deg_sh.at[dstv_t], add=True)

        plsc.subcore_barrier()
        _write_out(sid, cid, s_sh, s_out)
        _write_out(sid, cid, deg_sh, deg_out)

    return functools.partial(
        pl.kernel, mesh=_MESH,
        out_type=(jax.ShapeDtypeStruct((NC, N, DE), jnp.float32),
                  jax.ShapeDtypeStruct((NC, N, DE), jnp.float32)),
        scratch_types=scratch, compiler_params=_SC_PARAMS)(body)


_spmm = _make_spmm()
_sdeg = _make_sdeg()


def _make_mlp(final_relu: bool, with_sd_inputs: bool):
    R = 2000

    def body(agg_ref, h_ref, s_ref, d_ref, a_ref, b_ref, v_ref, u_ref,
             w2_ref, b2_ref, o_ref):
        z = agg_ref[0] + agg_ref[1] + h_ref[...]
        sarr = s_ref[0] + s_ref[1]
        darr = d_ref[0] + d_ref[1]
        dcol = darr[:, :1]
        pre = (jnp.dot(z, a_ref[...], preferred_element_type=jnp.float32)
               + jnp.dot(sarr, b_ref[...], preferred_element_type=jnp.float32)
               + dcol * v_ref[...] + u_ref[...])
        t = jnp.maximum(pre, 0.0)
        out = jnp.dot(t, w2_ref[...], preferred_element_type=jnp.float32) + b2_ref[...]
        if final_relu:
            out = jnp.maximum(out, 0.0)
        o_ref[...] = out

    grid = (N // R,)
    in_specs = [
        pl.BlockSpec((NC, R, D), lambda i: (0, i, 0)),
        pl.BlockSpec((R, D), lambda i: (i, 0)),
        pl.BlockSpec((NC, R, DE), lambda i: (0, i, 0)),
        pl.BlockSpec((NC, R, DE), lambda i: (0, i, 0)),
        pl.BlockSpec((D, 2 * D), lambda i: (0, 0)),
        pl.BlockSpec((DE, 2 * D), lambda i: (0, 0)),
        pl.BlockSpec((1, 2 * D), lambda i: (0, 0)),
        pl.BlockSpec((1, 2 * D), lambda i: (0, 0)),
        pl.BlockSpec((2 * D, D), lambda i: (0, 0)),
        pl.BlockSpec((1, D), lambda i: (0, 0)),
    ]
    return pl.pallas_call(
        body,
        grid=grid,
        in_specs=in_specs,
        out_specs=pl.BlockSpec((R, D), lambda i: (i, 0)),
        out_shape=jax.ShapeDtypeStruct((N, D), jnp.float32),
    )


_mlp0 = _make_mlp(final_relu=True, with_sd_inputs=True)
_mlp1 = _make_mlp(final_relu=False, with_sd_inputs=True)

_SCALE = 1.0 / np.sqrt(1.0 + EPS)


def kernel(x, edge_index, edge_attr, self_loop_index, self_loop_type,
           W_enc0, b_enc0, W1_0, b1_0, gamma0, beta0, W2_0, b2_0,
           W_enc1, b_enc1, W1_1, b1_1, gamma1, beta1, W2_1, b2_1):
    sl_row = ((jnp.arange(DE) == self_loop_index).astype(jnp.float32)
              * jnp.asarray(self_loop_type, jnp.float32))

    def fold(W1, b1, gamma, beta):
        g = gamma * _SCALE
        return W1 * g[None, :], b1 * g + beta

    W1f0, b1f0 = fold(W1_0, b1_0, gamma0, beta0)
    A0 = W1f0
    B0 = W_enc0 @ W1f0
    v0 = (b_enc0 @ W1f0)[None, :]
    u0 = ((sl_row @ W_enc0 + b_enc0) @ W1f0 + b1f0)[None, :]

    W1f1, b1f1 = fold(W1_1, b1_1, gamma1, beta1)
    A1 = W1f1[:D]
    Wb = W1f1[D:]
    B1 = W_enc1 @ Wb
    v1 = (b_enc1 @ Wb)[None, :]
    u1 = ((sl_row @ W_enc1 + b_enc1) @ Wb + b1f1)[None, :]

    ei_lin = edge_index.reshape(2 * E)
    (aggx,) = _spmm(x, ei_lin)
    ei_lin2, aggx = lax.optimization_barrier((ei_lin, aggx))
    S, deg = _sdeg(ei_lin2, edge_attr)
    h0 = _mlp0(aggx, x, S, deg, A0, B0, v0, u0, W2_0, b2_0[None, :])
    (aggh,) = _spmm(h0, ei_lin)
    h1 = _mlp1(aggh, h0, S, deg, A1, B1, v1, u1, W2_1, b2_1[None, :])
    return h1


# speedup vs baseline: 1.2422x; 1.0710x over previous
import functools

import jax
import jax.numpy as jnp
import numpy as np
from jax import lax
from jax.experimental import pallas as pl
from jax.experimental.pallas import tpu as pltpu
from jax.experimental.pallas import tpu_sc as plsc

N = 10000
E = 320000
D = 128
DE = 16
EPS = 1e-05

NC = 2
NS = 16
NW = NC * NS
EPW = E // NW
K = 128
NFULL = EPW // K
TE = EPW - NFULL * K
KA = 104
NFA = EPW // KA
TEA = EPW - NFA * KA
RO = 624
ZR = 48
TAIL = N - NS * RO

_MESH = plsc.VectorSubcoreMesh(core_axis_name="c", subcore_axis_name="s")
_SC_PARAMS = pltpu.CompilerParams(use_tc_tiling_on_sc=False)


def _zero_fill(zbuf, width):
    zv = jnp.zeros((16,), jnp.float32)

    @pl.loop(0, ZR)
    def _(i):
        @pl.loop(0, width // 16)
        def _(j):
            zbuf[i, pl.ds(j * 16, 16)] = zv


def _zero_shared(sid, zbuf, sh):
    rbase = sid * RO
    for r in range(RO // ZR):
        pltpu.sync_copy(zbuf, sh.at[pl.ds(rbase + r * ZR, ZR)])

    @pl.when(sid == NS - 1)
    def _():
        pltpu.sync_copy(zbuf.at[pl.ds(0, TAIL)], sh.at[pl.ds(NS * RO, TAIL)])


def _write_out(sid, cid, sh, out):
    rbase = sid * RO
    pltpu.sync_copy(sh.at[pl.ds(rbase, RO)], out.at[cid, pl.ds(rbase, RO)])

    @pl.when(sid == NS - 1)
    def _():
        pltpu.sync_copy(sh.at[pl.ds(NS * RO, TAIL)], out.at[cid, pl.ds(NS * RO, TAIL)])


def _make_spmm():
    scratch = [
        [pltpu.VMEM((KA,), jnp.int32)] * 3,
        [pltpu.VMEM((KA,), jnp.int32)] * 3,
        [pltpu.VMEM((KA, D), jnp.float32)] * 3,
        pltpu.VMEM((TEA,), jnp.int32),
        pltpu.VMEM((TEA,), jnp.int32),
        pltpu.VMEM((TEA, D), jnp.float32),
        pltpu.VMEM((ZR, D), jnp.float32),
        pltpu.VMEM_SHARED((N, D), jnp.float32),
        [pltpu.SemaphoreType.DMA] * 3,
        [pltpu.SemaphoreType.DMA] * 3,
        [pltpu.SemaphoreType.DMA] * 3,
        pltpu.SemaphoreType.DMA,
    ]

    def body(h_hbm, ei_hbm, agg_out, dstv, srcv, rows, dstv_t, srcv_t, rows_t,
             zbuf, agg_sh, sem_i, sem_g, sem_s, sem_t):
        cid = lax.axis_index("c")
        sid = lax.axis_index("s")
        _zero_fill(zbuf, D)
        _zero_shared(sid, zbuf, agg_sh)
        plsc.subcore_barrier()

        ebase = (sid * NC + cid) * EPW

        def start_inputs(c, b):
            off = ebase + c * KA
            pltpu.async_copy(ei_hbm.at[pl.ds(off, KA)], dstv[b], sem_i[b])
            pltpu.async_copy(ei_hbm.at[pl.ds(E + off, KA)], srcv[b], sem_i[b])

        def wait_inputs(b):
            pltpu.make_async_copy(ei_hbm.at[pl.ds(0, KA)], dstv[b], sem_i[b]).wait()
            pltpu.make_async_copy(ei_hbm.at[pl.ds(0, KA)], srcv[b], sem_i[b]).wait()

        def start_gather(b):
            pltpu.async_copy(h_hbm.at[srcv[b]], rows[b], sem_g[b])

        def wait_gather(b):
            pltpu.make_async_copy(h_hbm.at[srcv[b]], rows[b], sem_g[b]).wait()

        def start_scatter(b):
            pltpu.async_copy(rows[b], agg_sh.at[dstv[b]], sem_s[b], add=True)

        def wait_scatter(b):
            pltpu.make_async_copy(rows[b], agg_sh.at[dstv[b]], sem_s[b]).wait()

        start_inputs(0, 0)
        start_inputs(1, 1)
        start_inputs(2, 2)
        wait_inputs(0)
        start_gather(0)

        @pl.loop(0, NFA - 3, step=3)
        def _(j):
            for b in (0, 1, 2):
                y = j + b
                bn, bp = (b + 1) % 3, (b + 2) % 3
                wait_inputs(bn)
                if b == 0:
                    @pl.when(y >= 1)
                    def _():
                        wait_scatter(bp)
                        start_inputs(y + 2, bp)
                else:
                    wait_scatter(bp)
                    start_inputs(y + 2, bp)
                start_gather(bn)
                wait_gather(b)
                start_scatter(b)

        y = NFA - 3
        b, bn, bp = y % 3, (y + 1) % 3, (y + 2) % 3
        wait_inputs(bn)
        wait_scatter(bp)
        start_inputs(y + 2, bp)
        start_gather(bn)
        wait_gather(b)
        start_scatter(b)

        y = NFA - 2
        b, bn, bp = y % 3, (y + 1) % 3, (y + 2) % 3
        wait_inputs(bn)
        wait_scatter(bp)
        start_gather(bn)
        wait_gather(b)
        start_scatter(b)

        y = NFA - 1
        b, bn, bp = y % 3, (y + 1) % 3, (y + 2) % 3
        wait_scatter(bp)
        wait_gather(b)
        start_scatter(b)

        toff = ebase + NFA * KA
        pltpu.sync_copy(ei_hbm.at[pl.ds(toff, TEA)], dstv_t)
        pltpu.sync_copy(ei_hbm.at[pl.ds(E + toff, TEA)], srcv_t)
        pltpu.async_copy(h_hbm.at[srcv_t], rows_t, sem_t).wait()
        wait_scatter((NFA - 1) % 3)
        pltpu.sync_copy(rows_t, agg_sh.at[dstv_t], add=True)

        plsc.subcore_barrier()
        _write_out(sid, cid, agg_sh, agg_out)

    return functools.partial(
        pl.kernel, mesh=_MESH,
        out_type=(jax.ShapeDtypeStruct((NC, N, D), jnp.float32),),
        scratch_types=scratch, compiler_params=_SC_PARAMS)(body)


def _make_sdeg():
    scratch = [
        [pltpu.VMEM((K,), jnp.int32)] * 2,
        [pltpu.VMEM((K, DE), jnp.float32)] * 2,
        pltpu.VMEM((K, DE), jnp.float32),
        pltpu.VMEM((TE,), jnp.int32),
        pltpu.VMEM((TE, DE), jnp.float32),
        pltpu.VMEM((TE, DE), jnp.float32),
        pltpu.VMEM((ZR, DE), jnp.float32),
        pltpu.VMEM_SHARED((N, DE), jnp.float32),
        pltpu.VMEM_SHARED((N, DE), jnp.float32),
        [pltpu.SemaphoreType.DMA] * 2,
    ]

    def body(ei_hbm, ea_hbm, s_out, deg_out, dstv, eav, ones, dstv_t, eav_t,
             ones_t, zbuf16, s_sh, deg_sh, sem_i):
        cid = lax.axis_index("c")
        sid = lax.axis_index("s")
        ov = jnp.ones((16,), jnp.float32)

        @pl.loop(0, K)
        def _(i):
            ones[i, pl.ds(0, 16)] = ov

        @pl.loop(0, TE)
        def _(i):
            ones_t[i, pl.ds(0, 16)] = ov

        _zero_fill(zbuf16, DE)
        _zero_shared(sid, zbuf16, s_sh)
        _zero_shared(sid, zbuf16, deg_sh)
        plsc.subcore_barrier()

        ebase = (sid * NC + cid) * EPW

        def start_inputs(c, b):
            off = ebase + c * K
            pltpu.async_copy(ei_hbm.at[pl.ds(off, K)], dstv[b], sem_i[b])
            pltpu.async_copy(ea_hbm.at[pl.ds(off, K)], eav[b], sem_i[b])

        def wait_inputs(b):
            pltpu.make_async_copy(ei_hbm.at[pl.ds(0, K)], dstv[b], sem_i[b]).wait()
            pltpu.make_async_copy(ea_hbm.at[pl.ds(0, K)], eav[b], sem_i[b]).wait()

        def scatter(b):
            pltpu.sync_copy(eav[b], s_sh.at[dstv[b]], add=True)
            pltpu.sync_copy(ones, deg_sh.at[dstv[b]], add=True)

        start_inputs(0, 0)
        start_inputs(1, 1)

        @pl.loop(0, NFULL - 2, step=2)
        def _(j):
            for b in (0, 1):
                c = j + b
                wait_inputs(b)
                scatter(b)
                start_inputs(c + 2, b)

        wait_inputs(0)
        scatter(0)
        wait_inputs(1)
        scatter(1)

        toff = ebase + NFULL * K
        pltpu.sync_copy(ei_hbm.at[pl.ds(toff, TE)], dstv_t)
        pltpu.sync_copy(ea_hbm.at[pl.ds(toff, TE)], eav_t)
        pltpu.sync_copy(eav_t, s_sh.at[dstv_t], add=True)
        pltpu.sync_copy(ones_t, deg_sh.at[dstv_t], add=True)

        plsc.subcore_barrier()
        _write_out(sid, cid, s_sh, s_out)
        _write_out(sid, cid, deg_sh, deg_out)

    return functools.partial(
        pl.kernel, mesh=_MESH,
        out_type=(jax.ShapeDtypeStruct((NC, N, DE), jnp.float32),
                  jax.ShapeDtypeStruct((NC, N, DE), jnp.float32)),
        scratch_types=scratch, compiler_params=_SC_PARAMS)(body)


_spmm = _make_spmm()
_sdeg = _make_sdeg()


def _make_mlp(final_relu: bool, with_sd_inputs: bool):
    R = 2000

    def body(agg_ref, h_ref, s_ref, d_ref, a_ref, b_ref, v_ref, u_ref,
             w2_ref, b2_ref, o_ref):
        z = agg_ref[0] + agg_ref[1] + h_ref[...]
        sarr = s_ref[0] + s_ref[1]
        darr = d_ref[0] + d_ref[1]
        dcol = darr[:, :1]
        pre = (jnp.dot(z, a_ref[...], preferred_element_type=jnp.float32)
               + jnp.dot(sarr, b_ref[...], preferred_element_type=jnp.float32)
               + dcol * v_ref[...] + u_ref[...])
        t = jnp.maximum(pre, 0.0)
        out = jnp.dot(t, w2_ref[...], preferred_element_type=jnp.float32) + b2_ref[...]
        if final_relu:
            out = jnp.maximum(out, 0.0)
        o_ref[...] = out

    grid = (N // R,)
    in_specs = [
        pl.BlockSpec((NC, R, D), lambda i: (0, i, 0)),
        pl.BlockSpec((R, D), lambda i: (i, 0)),
        pl.BlockSpec((NC, R, DE), lambda i: (0, i, 0)),
        pl.BlockSpec((NC, R, DE), lambda i: (0, i, 0)),
        pl.BlockSpec((D, 2 * D), lambda i: (0, 0)),
        pl.BlockSpec((DE, 2 * D), lambda i: (0, 0)),
        pl.BlockSpec((1, 2 * D), lambda i: (0, 0)),
        pl.BlockSpec((1, 2 * D), lambda i: (0, 0)),
        pl.BlockSpec((2 * D, D), lambda i: (0, 0)),
        pl.BlockSpec((1, D), lambda i: (0, 0)),
    ]
    return pl.pallas_call(
        body,
        grid=grid,
        in_specs=in_specs,
        out_specs=pl.BlockSpec((R, D), lambda i: (i, 0)),
        out_shape=jax.ShapeDtypeStruct((N, D), jnp.float32),
    )


_mlp0 = _make_mlp(final_relu=True, with_sd_inputs=True)
_mlp1 = _make_mlp(final_relu=False, with_sd_inputs=True)

_SCALE = 1.0 / np.sqrt(1.0 + EPS)


def kernel(x, edge_index, edge_attr, self_loop_index, self_loop_type,
           W_enc0, b_enc0, W1_0, b1_0, gamma0, beta0, W2_0, b2_0,
           W_enc1, b_enc1, W1_1, b1_1, gamma1, beta1, W2_1, b2_1):
    sl_row = ((jnp.arange(DE) == self_loop_index).astype(jnp.float32)
              * jnp.asarray(self_loop_type, jnp.float32))

    def fold(W1, b1, gamma, beta):
        g = gamma * _SCALE
        return W1 * g[None, :], b1 * g + beta

    W1f0, b1f0 = fold(W1_0, b1_0, gamma0, beta0)
    A0 = W1f0
    B0 = W_enc0 @ W1f0
    v0 = (b_enc0 @ W1f0)[None, :]
    u0 = ((sl_row @ W_enc0 + b_enc0) @ W1f0 + b1f0)[None, :]

    W1f1, b1f1 = fold(W1_1, b1_1, gamma1, beta1)
    A1 = W1f1[:D]
    Wb = W1f1[D:]
    B1 = W_enc1 @ Wb
    v1 = (b_enc1 @ Wb)[None, :]
    u1 = ((sl_row @ W_enc1 + b_enc1) @ Wb + b1f1)[None, :]

    ei_lin = edge_index.reshape(2 * E)
    (aggx,) = _spmm(x, ei_lin)
    ei_lin2, aggx = lax.optimization_barrier((ei_lin, aggx))
    S, deg = _sdeg(ei_lin2, edge_attr)
    h0 = _mlp0(aggx, x, S, deg, A0, B0, v0, u0, W2_0, b2_0[None, :])
    (aggh,) = _spmm(h0, ei_lin)
    h1 = _mlp1(aggh, h0, S, deg, A1, B1, v1, u1, W2_1, b2_1[None, :])
    return h1


# speedup vs baseline: 1.2595x; 1.0139x over previous
import functools

import jax
import jax.numpy as jnp
import numpy as np
from jax import lax
from jax.experimental import pallas as pl
from jax.experimental.pallas import tpu as pltpu
from jax.experimental.pallas import tpu_sc as plsc

N = 10000
E = 320000
D = 128
DE = 16
EPS = 1e-05

NC = 2
NS = 16
NW = NC * NS
EPW = E // NW
K = 128
NFULL = EPW // K
TE = EPW - NFULL * K
KA = 104
NFA = EPW // KA
TEA = EPW - NFA * KA
RO = 624
ZR = 48
TAIL = N - NS * RO

_MESH = plsc.VectorSubcoreMesh(core_axis_name="c", subcore_axis_name="s")
_SC_PARAMS = pltpu.CompilerParams(use_tc_tiling_on_sc=False)


def _zero_fill(zbuf, width):
    zv = jnp.zeros((16,), jnp.float32)

    @pl.loop(0, ZR)
    def _(i):
        @pl.loop(0, width // 16)
        def _(j):
            zbuf[i, pl.ds(j * 16, 16)] = zv


def _zero_fill_bf16(zbuf, width):
    zv = jnp.zeros((32,), jnp.bfloat16)

    @pl.loop(0, ZR)
    def _(i):
        @pl.loop(0, width // 32)
        def _(j):
            zbuf[i, pl.ds(j * 32, 32)] = zv


def _zero_shared(sid, zbuf, sh):
    rbase = sid * RO
    for r in range(RO // ZR):
        pltpu.sync_copy(zbuf, sh.at[pl.ds(rbase + r * ZR, ZR)])

    @pl.when(sid == NS - 1)
    def _():
        pltpu.sync_copy(zbuf.at[pl.ds(0, TAIL)], sh.at[pl.ds(NS * RO, TAIL)])


def _write_out(sid, cid, sh, out):
    rbase = sid * RO
    pltpu.sync_copy(sh.at[pl.ds(rbase, RO)], out.at[cid, pl.ds(rbase, RO)])

    @pl.when(sid == NS - 1)
    def _():
        pltpu.sync_copy(sh.at[pl.ds(NS * RO, TAIL)], out.at[cid, pl.ds(NS * RO, TAIL)])


def _make_spmm():
    scratch = [
        [pltpu.VMEM((KA,), jnp.int32)] * 3,
        [pltpu.VMEM((KA,), jnp.int32)] * 3,
        [pltpu.VMEM((KA, D), jnp.bfloat16)] * 3,
        pltpu.VMEM((TEA,), jnp.int32),
        pltpu.VMEM((TEA,), jnp.int32),
        pltpu.VMEM((TEA, D), jnp.bfloat16),
        pltpu.VMEM((ZR, D), jnp.bfloat16),
        pltpu.VMEM_SHARED((N, D), jnp.bfloat16),
        [pltpu.SemaphoreType.DMA] * 3,
        [pltpu.SemaphoreType.DMA] * 3,
        [pltpu.SemaphoreType.DMA] * 3,
        pltpu.SemaphoreType.DMA,
    ]

    def body(h_hbm, ei_hbm, agg_out, dstv, srcv, rows, dstv_t, srcv_t, rows_t,
             zbuf, agg_sh, sem_i, sem_g, sem_s, sem_t):
        cid = lax.axis_index("c")
        sid = lax.axis_index("s")
        _zero_fill_bf16(zbuf, D)
        _zero_shared(sid, zbuf, agg_sh)
        plsc.subcore_barrier()

        ebase = (sid * NC + cid) * EPW

        def start_inputs(c, b):
            off = ebase + c * KA
            pltpu.async_copy(ei_hbm.at[pl.ds(off, KA)], dstv[b], sem_i[b])
            pltpu.async_copy(ei_hbm.at[pl.ds(E + off, KA)], srcv[b], sem_i[b])

        def wait_inputs(b):
            pltpu.make_async_copy(ei_hbm.at[pl.ds(0, KA)], dstv[b], sem_i[b]).wait()
            pltpu.make_async_copy(ei_hbm.at[pl.ds(0, KA)], srcv[b], sem_i[b]).wait()

        def start_gather(b):
            pltpu.async_copy(h_hbm.at[srcv[b]], rows[b], sem_g[b])

        def wait_gather(b):
            pltpu.make_async_copy(h_hbm.at[srcv[b]], rows[b], sem_g[b]).wait()

        def start_scatter(b):
            pltpu.async_copy(rows[b], agg_sh.at[dstv[b]], sem_s[b], add=True)

        def wait_scatter(b):
            pltpu.make_async_copy(rows[b], agg_sh.at[dstv[b]], sem_s[b]).wait()

        start_inputs(0, 0)
        start_inputs(1, 1)
        start_inputs(2, 2)
        wait_inputs(0)
        start_gather(0)

        @pl.loop(0, NFA - 3, step=3)
        def _(j):
            for b in (0, 1, 2):
                y = j + b
                bn, bp = (b + 1) % 3, (b + 2) % 3
                wait_inputs(bn)
                if b == 0:
                    @pl.when(y >= 1)
                    def _():
                        wait_scatter(bp)
                        start_inputs(y + 2, bp)
                else:
                    wait_scatter(bp)
                    start_inputs(y + 2, bp)
                start_gather(bn)
                wait_gather(b)
                start_scatter(b)

        y = NFA - 3
        b, bn, bp = y % 3, (y + 1) % 3, (y + 2) % 3
        wait_inputs(bn)
        wait_scatter(bp)
        start_inputs(y + 2, bp)
        start_gather(bn)
        wait_gather(b)
        start_scatter(b)

        y = NFA - 2
        b, bn, bp = y % 3, (y + 1) % 3, (y + 2) % 3
        wait_inputs(bn)
        wait_scatter(bp)
        start_gather(bn)
        wait_gather(b)
        start_scatter(b)

        y = NFA - 1
        b, bn, bp = y % 3, (y + 1) % 3, (y + 2) % 3
        wait_scatter(bp)
        wait_gather(b)
        start_scatter(b)

        toff = ebase + NFA * KA
        pltpu.sync_copy(ei_hbm.at[pl.ds(toff, TEA)], dstv_t)
        pltpu.sync_copy(ei_hbm.at[pl.ds(E + toff, TEA)], srcv_t)
        pltpu.async_copy(h_hbm.at[srcv_t], rows_t, sem_t).wait()
        wait_scatter((NFA - 1) % 3)
        pltpu.sync_copy(rows_t, agg_sh.at[dstv_t], add=True)

        plsc.subcore_barrier()
        _write_out(sid, cid, agg_sh, agg_out)

    return functools.partial(
        pl.kernel, mesh=_MESH,
        out_type=(jax.ShapeDtypeStruct((NC, N, D), jnp.bfloat16),),
        scratch_types=scratch, compiler_params=_SC_PARAMS)(body)


def _make_sdeg():
    scratch = [
        [pltpu.VMEM((K,), jnp.int32)] * 2,
        [pltpu.VMEM((K, DE), jnp.float32)] * 2,
        pltpu.VMEM((K, DE), jnp.float32),
        pltpu.VMEM((TE,), jnp.int32),
        pltpu.VMEM((TE, DE), jnp.float32),
        pltpu.VMEM((TE, DE), jnp.float32),
        pltpu.VMEM((ZR, DE), jnp.float32),
        pltpu.VMEM_SHARED((N, DE), jnp.float32),
        pltpu.VMEM_SHARED((N, DE), jnp.float32),
        [pltpu.SemaphoreType.DMA] * 2,
    ]

    def body(ei_hbm, ea_hbm, s_out, deg_out, dstv, eav, ones, dstv_t, eav_t,
             ones_t, zbuf16, s_sh, deg_sh, sem_i):
        cid = lax.axis_index("c")
        sid = lax.axis_index("s")
        ov = jnp.ones((16,), jnp.float32)

        @pl.loop(0, K)
        def _(i):
            ones[i, pl.ds(0, 16)] = ov

        @pl.loop(0, TE)
        def _(i):
            ones_t[i, pl.ds(0, 16)] = ov

        _zero_fill(zbuf16, DE)
        _zero_shared(sid, zbuf16, s_sh)
        _zero_shared(sid, zbuf16, deg_sh)
        plsc.subcore_barrier()

        ebase = (sid * NC + cid) * EPW

        def start_inputs(c, b):
            off = ebase + c * K
            pltpu.async_copy(ei_hbm.at[pl.ds(off, K)], dstv[b], sem_i[b])
            pltpu.async_copy(ea_hbm.at[pl.ds(off, K)], eav[b], sem_i[b])

        def wait_inputs(b):
            pltpu.make_async_copy(ei_hbm.at[pl.ds(0, K)], dstv[b], sem_i[b]).wait()
            pltpu.make_async_copy(ea_hbm.at[pl.ds(0, K)], eav[b], sem_i[b]).wait()

        def scatter(b):
            pltpu.sync_copy(eav[b], s_sh.at[dstv[b]], add=True)
            pltpu.sync_copy(ones, deg_sh.at[dstv[b]], add=True)

        start_inputs(0, 0)
        start_inputs(1, 1)

        @pl.loop(0, NFULL - 2, step=2)
        def _(j):
            for b in (0, 1):
                c = j + b
                wait_inputs(b)
                scatter(b)
                start_inputs(c + 2, b)

        wait_inputs(0)
        scatter(0)
        wait_inputs(1)
        scatter(1)

        toff = ebase + NFULL * K
        pltpu.sync_copy(ei_hbm.at[pl.ds(toff, TE)], dstv_t)
        pltpu.sync_copy(ea_hbm.at[pl.ds(toff, TE)], eav_t)
        pltpu.sync_copy(eav_t, s_sh.at[dstv_t], add=True)
        pltpu.sync_copy(ones_t, deg_sh.at[dstv_t], add=True)

        plsc.subcore_barrier()
        _write_out(sid, cid, s_sh, s_out)
        _write_out(sid, cid, deg_sh, deg_out)

    return functools.partial(
        pl.kernel, mesh=_MESH,
        out_type=(jax.ShapeDtypeStruct((NC, N, DE), jnp.float32),
                  jax.ShapeDtypeStruct((NC, N, DE), jnp.float32)),
        scratch_types=scratch, compiler_params=_SC_PARAMS)(body)


_spmm = _make_spmm()
_sdeg = _make_sdeg()


def _make_mlp(final_relu: bool, with_sd_inputs: bool):
    R = 2000

    def body(agg_ref, h_ref, s_ref, d_ref, a_ref, b_ref, v_ref, u_ref,
             w2_ref, b2_ref, o_ref):
        z = (agg_ref[0].astype(jnp.float32) + agg_ref[1].astype(jnp.float32)
             + h_ref[...])
        sarr = s_ref[0] + s_ref[1]
        darr = d_ref[0] + d_ref[1]
        dcol = darr[:, :1]
        pre = (jnp.dot(z, a_ref[...], preferred_element_type=jnp.float32)
               + jnp.dot(sarr, b_ref[...], preferred_element_type=jnp.float32)
               + dcol * v_ref[...] + u_ref[...])
        t = jnp.maximum(pre, 0.0)
        out = jnp.dot(t, w2_ref[...], preferred_element_type=jnp.float32) + b2_ref[...]
        if final_relu:
            out = jnp.maximum(out, 0.0)
        o_ref[...] = out

    grid = (N // R,)
    in_specs = [
        pl.BlockSpec((NC, R, D), lambda i: (0, i, 0)),
        pl.BlockSpec((R, D), lambda i: (i, 0)),
        pl.BlockSpec((NC, R, DE), lambda i: (0, i, 0)),
        pl.BlockSpec((NC, R, DE), lambda i: (0, i, 0)),
        pl.BlockSpec((D, 2 * D), lambda i: (0, 0)),
        pl.BlockSpec((DE, 2 * D), lambda i: (0, 0)),
        pl.BlockSpec((1, 2 * D), lambda i: (0, 0)),
        pl.BlockSpec((1, 2 * D), lambda i: (0, 0)),
        pl.BlockSpec((2 * D, D), lambda i: (0, 0)),
        pl.BlockSpec((1, D), lambda i: (0, 0)),
    ]
    return pl.pallas_call(
        body,
        grid=grid,
        in_specs=in_specs,
        out_specs=pl.BlockSpec((R, D), lambda i: (i, 0)),
        out_shape=jax.ShapeDtypeStruct((N, D), jnp.float32),
    )


_mlp0 = _make_mlp(final_relu=True, with_sd_inputs=True)
_mlp1 = _make_mlp(final_relu=False, with_sd_inputs=True)

_SCALE = 1.0 / np.sqrt(1.0 + EPS)


def kernel(x, edge_index, edge_attr, self_loop_index, self_loop_type,
           W_enc0, b_enc0, W1_0, b1_0, gamma0, beta0, W2_0, b2_0,
           W_enc1, b_enc1, W1_1, b1_1, gamma1, beta1, W2_1, b2_1):
    sl_row = ((jnp.arange(DE) == self_loop_index).astype(jnp.float32)
              * jnp.asarray(self_loop_type, jnp.float32))

    def fold(W1, b1, gamma, beta):
        g = gamma * _SCALE
        return W1 * g[None, :], b1 * g + beta

    W1f0, b1f0 = fold(W1_0, b1_0, gamma0, beta0)
    A0 = W1f0
    B0 = W_enc0 @ W1f0
    v0 = (b_enc0 @ W1f0)[None, :]
    u0 = ((sl_row @ W_enc0 + b_enc0) @ W1f0 + b1f0)[None, :]

    W1f1, b1f1 = fold(W1_1, b1_1, gamma1, beta1)
    A1 = W1f1[:D]
    Wb = W1f1[D:]
    B1 = W_enc1 @ Wb
    v1 = (b_enc1 @ Wb)[None, :]
    u1 = ((sl_row @ W_enc1 + b_enc1) @ Wb + b1f1)[None, :]

    ei_lin = edge_index.reshape(2 * E)
    (aggx,) = _spmm(x.astype(jnp.bfloat16), ei_lin)
    ei_lin2, aggx = lax.optimization_barrier((ei_lin, aggx))
    S, deg = _sdeg(ei_lin2, edge_attr)
    h0 = _mlp0(aggx, x, S, deg, A0, B0, v0, u0, W2_0, b2_0[None, :])
    (aggh,) = _spmm(h0.astype(jnp.bfloat16), ei_lin)
    h1 = _mlp1(aggh, h0, S, deg, A1, B1, v1, u1, W2_1, b2_1[None, :])
    return h1


# speedup vs baseline: 1.2758x; 1.0130x over previous
import functools

import jax
import jax.numpy as jnp
import numpy as np
from jax import lax
from jax.experimental import pallas as pl
from jax.experimental.pallas import tpu as pltpu
from jax.experimental.pallas import tpu_sc as plsc

N = 10000
E = 320000
D = 128
DE = 16
EPS = 1e-05

NC = 2
NS = 16
NW = NC * NS
EPW = E // NW
K = 128
NFULL = EPW // K
TE = EPW - NFULL * K
KA = 128
NFA = EPW // KA
TEA = EPW - NFA * KA
RO = 624
ZR = 48
TAIL = N - NS * RO

_MESH = plsc.VectorSubcoreMesh(core_axis_name="c", subcore_axis_name="s")
_SC_PARAMS = pltpu.CompilerParams(use_tc_tiling_on_sc=False)


def _zero_fill(zbuf, width):
    zv = jnp.zeros((16,), jnp.float32)

    @pl.loop(0, ZR)
    def _(i):
        @pl.loop(0, width // 16)
        def _(j):
            zbuf[i, pl.ds(j * 16, 16)] = zv


def _zero_fill_bf16(zbuf, width):
    zv = jnp.zeros((32,), jnp.bfloat16)

    @pl.loop(0, ZR)
    def _(i):
        @pl.loop(0, width // 32)
        def _(j):
            zbuf[i, pl.ds(j * 32, 32)] = zv


def _zero_shared(sid, zbuf, sh):
    rbase = sid * RO
    for r in range(RO // ZR):
        pltpu.sync_copy(zbuf, sh.at[pl.ds(rbase + r * ZR, ZR)])

    @pl.when(sid == NS - 1)
    def _():
        pltpu.sync_copy(zbuf.at[pl.ds(0, TAIL)], sh.at[pl.ds(NS * RO, TAIL)])


def _write_out(sid, cid, sh, out):
    rbase = sid * RO
    pltpu.sync_copy(sh.at[pl.ds(rbase, RO)], out.at[cid, pl.ds(rbase, RO)])

    @pl.when(sid == NS - 1)
    def _():
        pltpu.sync_copy(sh.at[pl.ds(NS * RO, TAIL)], out.at[cid, pl.ds(NS * RO, TAIL)])


def _make_spmm():
    scratch = [
        [pltpu.VMEM((KA,), jnp.int32)] * 3,
        [pltpu.VMEM((KA,), jnp.int32)] * 3,
        [pltpu.VMEM((KA, D), jnp.bfloat16)] * 3,
        pltpu.VMEM((TEA,), jnp.int32),
        pltpu.VMEM((TEA,), jnp.int32),
        pltpu.VMEM((TEA, D), jnp.bfloat16),
        pltpu.VMEM((ZR, D), jnp.bfloat16),
        pltpu.VMEM_SHARED((N, D), jnp.bfloat16),
        [pltpu.SemaphoreType.DMA] * 3,
        [pltpu.SemaphoreType.DMA] * 3,
        [pltpu.SemaphoreType.DMA] * 3,
        pltpu.SemaphoreType.DMA,
    ]

    def body(h_hbm, ei_hbm, agg_out, dstv, srcv, rows, dstv_t, srcv_t, rows_t,
             zbuf, agg_sh, sem_i, sem_g, sem_s, sem_t):
        cid = lax.axis_index("c")
        sid = lax.axis_index("s")
        _zero_fill_bf16(zbuf, D)
        _zero_shared(sid, zbuf, agg_sh)
        plsc.subcore_barrier()

        ebase = (sid * NC + cid) * EPW

        def start_inputs(c, b):
            off = ebase + c * KA
            pltpu.async_copy(ei_hbm.at[pl.ds(off, KA)], dstv[b], sem_i[b])
            pltpu.async_copy(ei_hbm.at[pl.ds(E + off, KA)], srcv[b], sem_i[b])

        def wait_inputs(b):
            pltpu.make_async_copy(ei_hbm.at[pl.ds(0, KA)], dstv[b], sem_i[b]).wait()
            pltpu.make_async_copy(ei_hbm.at[pl.ds(0, KA)], srcv[b], sem_i[b]).wait()

        def start_gather(b):
            pltpu.async_copy(h_hbm.at[srcv[b]], rows[b], sem_g[b])

        def wait_gather(b):
            pltpu.make_async_copy(h_hbm.at[srcv[b]], rows[b], sem_g[b]).wait()

        def start_scatter(b):
            pltpu.async_copy(rows[b], agg_sh.at[dstv[b]], sem_s[b], add=True)

        def wait_scatter(b):
            pltpu.make_async_copy(rows[b], agg_sh.at[dstv[b]], sem_s[b]).wait()

        start_inputs(0, 0)
        start_inputs(1, 1)
        start_inputs(2, 2)
        wait_inputs(0)
        start_gather(0)

        @pl.loop(0, NFA - 3, step=3)
        def _(j):
            for b in (0, 1, 2):
                y = j + b
                bn, bp = (b + 1) % 3, (b + 2) % 3
                wait_inputs(bn)
                if b == 0:
                    @pl.when(y >= 1)
                    def _():
                        wait_scatter(bp)
                        start_inputs(y + 2, bp)
                else:
                    wait_scatter(bp)
                    start_inputs(y + 2, bp)
                start_gather(bn)
                wait_gather(b)
                start_scatter(b)

        y = NFA - 3
        b, bn, bp = y % 3, (y + 1) % 3, (y + 2) % 3
        wait_inputs(bn)
        wait_scatter(bp)
        start_inputs(y + 2, bp)
        start_gather(bn)
        wait_gather(b)
        start_scatter(b)

        y = NFA - 2
        b, bn, bp = y % 3, (y + 1) % 3, (y + 2) % 3
        wait_inputs(bn)
        wait_scatter(bp)
        start_gather(bn)
        wait_gather(b)
        start_scatter(b)

        y = NFA - 1
        b, bn, bp = y % 3, (y + 1) % 3, (y + 2) % 3
        wait_scatter(bp)
        wait_gather(b)
        start_scatter(b)

        toff = ebase + NFA * KA
        pltpu.sync_copy(ei_hbm.at[pl.ds(toff, TEA)], dstv_t)
        pltpu.sync_copy(ei_hbm.at[pl.ds(E + toff, TEA)], srcv_t)
        pltpu.async_copy(h_hbm.at[srcv_t], rows_t, sem_t).wait()
        wait_scatter((NFA - 1) % 3)
        pltpu.sync_copy(rows_t, agg_sh.at[dstv_t], add=True)

        plsc.subcore_barrier()
        _write_out(sid, cid, agg_sh, agg_out)

    return functools.partial(
        pl.kernel, mesh=_MESH,
        out_type=(jax.ShapeDtypeStruct((NC, N, D), jnp.bfloat16),),
        scratch_types=scratch, compiler_params=_SC_PARAMS)(body)


def _make_sdeg():
    scratch = [
        [pltpu.VMEM((K,), jnp.int32)] * 2,
        [pltpu.VMEM((K, DE), jnp.float32)] * 2,
        pltpu.VMEM((K, DE), jnp.float32),
        pltpu.VMEM((TE,), jnp.int32),
        pltpu.VMEM((TE, DE), jnp.float32),
        pltpu.VMEM((TE, DE), jnp.float32),
        pltpu.VMEM((ZR, DE), jnp.float32),
        pltpu.VMEM_SHARED((N, DE), jnp.float32),
        pltpu.VMEM_SHARED((N, DE), jnp.float32),
        [pltpu.SemaphoreType.DMA] * 2,
    ]

    def body(ei_hbm, ea_hbm, s_out, deg_out, dstv, eav, ones, dstv_t, eav_t,
             ones_t, zbuf16, s_sh, deg_sh, sem_i):
        cid = lax.axis_index("c")
        sid = lax.axis_index("s")
        ov = jnp.ones((16,), jnp.float32)

        @pl.loop(0, K)
        def _(i):
            ones[i, pl.ds(0, 16)] = ov

        @pl.loop(0, TE)
        def _(i):
            ones_t[i, pl.ds(0, 16)] = ov

        _zero_fill(zbuf16, DE)
        _zero_shared(sid, zbuf16, s_sh)
        _zero_shared(sid, zbuf16, deg_sh)
        plsc.subcore_barrier()

        ebase = (sid * NC + cid) * EPW

        def start_inputs(c, b):
            off = ebase + c * K
            pltpu.async_copy(ei_hbm.at[pl.ds(off, K)], dstv[b], sem_i[b])
            pltpu.async_copy(ea_hbm.at[pl.ds(off, K)], eav[b], sem_i[b])

        def wait_inputs(b):
            pltpu.make_async_copy(ei_hbm.at[pl.ds(0, K)], dstv[b], sem_i[b]).wait()
            pltpu.make_async_copy(ea_hbm.at[pl.ds(0, K)], eav[b], sem_i[b]).wait()

        def scatter(b):
            pltpu.sync_copy(eav[b], s_sh.at[dstv[b]], add=True)
            pltpu.sync_copy(ones, deg_sh.at[dstv[b]], add=True)

        start_inputs(0, 0)
        start_inputs(1, 1)

        @pl.loop(0, NFULL - 2, step=2)
        def _(j):
            for b in (0, 1):
                c = j + b
                wait_inputs(b)
                scatter(b)
                start_inputs(c + 2, b)

        wait_inputs(0)
        scatter(0)
        wait_inputs(1)
        scatter(1)

        toff = ebase + NFULL * K
        pltpu.sync_copy(ei_hbm.at[pl.ds(toff, TE)], dstv_t)
        pltpu.sync_copy(ea_hbm.at[pl.ds(toff, TE)], eav_t)
        pltpu.sync_copy(eav_t, s_sh.at[dstv_t], add=True)
        pltpu.sync_copy(ones_t, deg_sh.at[dstv_t], add=True)

        plsc.subcore_barrier()
        _write_out(sid, cid, s_sh, s_out)
        _write_out(sid, cid, deg_sh, deg_out)

    return functools.partial(
        pl.kernel, mesh=_MESH,
        out_type=(jax.ShapeDtypeStruct((NC, N, DE), jnp.float32),
                  jax.ShapeDtypeStruct((NC, N, DE), jnp.float32)),
        scratch_types=scratch, compiler_params=_SC_PARAMS)(body)


_spmm = _make_spmm()
_sdeg = _make_sdeg()


def _make_mlp(final_relu: bool, with_sd_inputs: bool):
    R = 2000

    def body(agg_ref, h_ref, s_ref, d_ref, a_ref, b_ref, v_ref, u_ref,
             w2_ref, b2_ref, o_ref):
        z = (agg_ref[0].astype(jnp.float32) + agg_ref[1].astype(jnp.float32)
             + h_ref[...])
        sarr = s_ref[0] + s_ref[1]
        darr = d_ref[0] + d_ref[1]
        dcol = darr[:, :1]
        pre = (jnp.dot(z, a_ref[...], preferred_element_type=jnp.float32)
               + jnp.dot(sarr, b_ref[...], preferred_element_type=jnp.float32)
               + dcol * v_ref[...] + u_ref[...])
        t = jnp.maximum(pre, 0.0)
        out = jnp.dot(t, w2_ref[...], preferred_element_type=jnp.float32) + b2_ref[...]
        if final_relu:
            out = jnp.maximum(out, 0.0)
        o_ref[...] = out

    grid = (N // R,)
    in_specs = [
        pl.BlockSpec((NC, R, D), lambda i: (0, i, 0)),
        pl.BlockSpec((R, D), lambda i: (i, 0)),
        pl.BlockSpec((NC, R, DE), lambda i: (0, i, 0)),
        pl.BlockSpec((NC, R, DE), lambda i: (0, i, 0)),
        pl.BlockSpec((D, 2 * D), lambda i: (0, 0)),
        pl.BlockSpec((DE, 2 * D), lambda i: (0, 0)),
        pl.BlockSpec((1, 2 * D), lambda i: (0, 0)),
        pl.BlockSpec((1, 2 * D), lambda i: (0, 0)),
        pl.BlockSpec((2 * D, D), lambda i: (0, 0)),
        pl.BlockSpec((1, D), lambda i: (0, 0)),
    ]
    return pl.pallas_call(
        body,
        grid=grid,
        in_specs=in_specs,
        out_specs=pl.BlockSpec((R, D), lambda i: (i, 0)),
        out_shape=jax.ShapeDtypeStruct((N, D), jnp.float32),
    )


_mlp0 = _make_mlp(final_relu=True, with_sd_inputs=True)
_mlp1 = _make_mlp(final_relu=False, with_sd_inputs=True)

_SCALE = 1.0 / np.sqrt(1.0 + EPS)


def kernel(x, edge_index, edge_attr, self_loop_index, self_loop_type,
           W_enc0, b_enc0, W1_0, b1_0, gamma0, beta0, W2_0, b2_0,
           W_enc1, b_enc1, W1_1, b1_1, gamma1, beta1, W2_1, b2_1):
    sl_row = ((jnp.arange(DE) == self_loop_index).astype(jnp.float32)
              * jnp.asarray(self_loop_type, jnp.float32))

    def fold(W1, b1, gamma, beta):
        g = gamma * _SCALE
        return W1 * g[None, :], b1 * g + beta

    W1f0, b1f0 = fold(W1_0, b1_0, gamma0, beta0)
    A0 = W1f0
    B0 = W_enc0 @ W1f0
    v0 = (b_enc0 @ W1f0)[None, :]
    u0 = ((sl_row @ W_enc0 + b_enc0) @ W1f0 + b1f0)[None, :]

    W1f1, b1f1 = fold(W1_1, b1_1, gamma1, beta1)
    A1 = W1f1[:D]
    Wb = W1f1[D:]
    B1 = W_enc1 @ Wb
    v1 = (b_enc1 @ Wb)[None, :]
    u1 = ((sl_row @ W_enc1 + b_enc1) @ Wb + b1f1)[None, :]

    ei_lin = edge_index.reshape(2 * E)
    (aggx,) = _spmm(x.astype(jnp.bfloat16), ei_lin)
    ei_lin2, aggx = lax.optimization_barrier((ei_lin, aggx))
    S, deg = _sdeg(ei_lin2, edge_attr)
    h0 = _mlp0(aggx, x, S, deg, A0, B0, v0, u0, W2_0, b2_0[None, :])
    (aggh,) = _spmm(h0.astype(jnp.bfloat16), ei_lin)
    h1 = _mlp1(aggh, h0, S, deg, A1, B1, v1, u1, W2_1, b2_1[None, :])
    return h1


# speedup vs baseline: 1.3136x; 1.0296x over previous
import functools

import jax
import jax.numpy as jnp
import numpy as np
from jax import lax
from jax.experimental import pallas as pl
from jax.experimental.pallas import tpu as pltpu
from jax.experimental.pallas import tpu_sc as plsc

N = 10000
E = 320000
D = 128
DE = 16
EPS = 1e-05

NC = 2
NS = 16
NW = NC * NS
EPW = E // NW
K = 128
NFULL = EPW // K
TE = EPW - NFULL * K
KA = 128
NFA = EPW // KA
TEA = EPW - NFA * KA
RO = 624
ZR = 48
TAIL = N - NS * RO

_MESH = plsc.VectorSubcoreMesh(core_axis_name="c", subcore_axis_name="s")
_SC_PARAMS = pltpu.CompilerParams(use_tc_tiling_on_sc=False)


def _zero_fill(zbuf, width):
    zv = jnp.zeros((16,), jnp.float32)

    @pl.loop(0, ZR)
    def _(i):
        @pl.loop(0, width // 16)
        def _(j):
            zbuf[i, pl.ds(j * 16, 16)] = zv


def _zero_fill_bf16(zbuf, width):
    zv = jnp.zeros((32,), jnp.bfloat16)

    @pl.loop(0, ZR)
    def _(i):
        @pl.loop(0, width // 32)
        def _(j):
            zbuf[i, pl.ds(j * 32, 32)] = zv


def _zero_shared(sid, zbuf, sh):
    rbase = sid * RO
    for r in range(RO // ZR):
        pltpu.sync_copy(zbuf, sh.at[pl.ds(rbase + r * ZR, ZR)])

    @pl.when(sid == NS - 1)
    def _():
        pltpu.sync_copy(zbuf.at[pl.ds(0, TAIL)], sh.at[pl.ds(NS * RO, TAIL)])


def _write_out(sid, cid, sh, out):
    rbase = sid * RO
    pltpu.sync_copy(sh.at[pl.ds(rbase, RO)], out.at[cid, pl.ds(rbase, RO)])

    @pl.when(sid == NS - 1)
    def _():
        pltpu.sync_copy(sh.at[pl.ds(NS * RO, TAIL)], out.at[cid, pl.ds(NS * RO, TAIL)])


def _make_spmm():
    scratch = [
        [pltpu.VMEM((KA,), jnp.int32)] * 3,
        [pltpu.VMEM((KA,), jnp.int32)] * 3,
        [pltpu.VMEM((KA, D), jnp.bfloat16)] * 3,
        pltpu.VMEM((TEA,), jnp.int32),
        pltpu.VMEM((TEA,), jnp.int32),
        pltpu.VMEM((TEA, D), jnp.bfloat16),
        pltpu.VMEM((ZR, D), jnp.bfloat16),
        pltpu.VMEM_SHARED((N, D), jnp.bfloat16),
        [pltpu.SemaphoreType.DMA] * 3,
        [pltpu.SemaphoreType.DMA] * 3,
        [pltpu.SemaphoreType.DMA] * 3,
        pltpu.SemaphoreType.DMA,
    ]

    def body(h_hbm, ei_hbm, agg_out, dstv, srcv, rows, dstv_t, srcv_t, rows_t,
             zbuf, agg_sh, sem_i, sem_g, sem_s, sem_t):
        cid = lax.axis_index("c")
        sid = lax.axis_index("s")
        _zero_fill_bf16(zbuf, D)
        _zero_shared(sid, zbuf, agg_sh)
        plsc.subcore_barrier()

        ebase = (sid * NC + cid) * EPW

        def start_inputs(c, b):
            off = ebase + c * KA
            pltpu.async_copy(ei_hbm.at[pl.ds(off, KA)], dstv[b], sem_i[b])
            pltpu.async_copy(ei_hbm.at[pl.ds(E + off, KA)], srcv[b], sem_i[b])

        def wait_inputs(b):
            pltpu.make_async_copy(ei_hbm.at[pl.ds(0, KA)], dstv[b], sem_i[b]).wait()
            pltpu.make_async_copy(ei_hbm.at[pl.ds(0, KA)], srcv[b], sem_i[b]).wait()

        def start_gather(b):
            pltpu.async_copy(h_hbm.at[srcv[b]], rows[b], sem_g[b])

        def wait_gather(b):
            pltpu.make_async_copy(h_hbm.at[srcv[b]], rows[b], sem_g[b]).wait()

        def start_scatter(b):
            pltpu.async_copy(rows[b], agg_sh.at[dstv[b]], sem_s[b], add=True)

        def wait_scatter(b):
            pltpu.make_async_copy(rows[b], agg_sh.at[dstv[b]], sem_s[b]).wait()

        start_inputs(0, 0)
        start_inputs(1, 1)
        start_inputs(2, 2)
        wait_inputs(0)
        start_gather(0)

        @pl.loop(0, NFA - 3, step=3)
        def _(j):
            for b in (0, 1, 2):
                y = j + b
                bn, bp = (b + 1) % 3, (b + 2) % 3
                wait_inputs(bn)
                if b == 0:
                    @pl.when(y >= 1)
                    def _():
                        wait_scatter(bp)
                        start_inputs(y + 2, bp)
                else:
                    wait_scatter(bp)
                    start_inputs(y + 2, bp)
                start_gather(bn)
                wait_gather(b)
                start_scatter(b)

        y = NFA - 3
        b, bn, bp = y % 3, (y + 1) % 3, (y + 2) % 3
        wait_inputs(bn)
        wait_scatter(bp)
        start_inputs(y + 2, bp)
        start_gather(bn)
        wait_gather(b)
        start_scatter(b)

        y = NFA - 2
        b, bn, bp = y % 3, (y + 1) % 3, (y + 2) % 3
        wait_inputs(bn)
        wait_scatter(bp)
        start_gather(bn)
        wait_gather(b)
        start_scatter(b)

        y = NFA - 1
        b, bn, bp = y % 3, (y + 1) % 3, (y + 2) % 3
        wait_scatter(bp)
        wait_gather(b)
        start_scatter(b)

        toff = ebase + NFA * KA
        pltpu.sync_copy(ei_hbm.at[pl.ds(toff, TEA)], dstv_t)
        pltpu.sync_copy(ei_hbm.at[pl.ds(E + toff, TEA)], srcv_t)
        pltpu.async_copy(h_hbm.at[srcv_t], rows_t, sem_t).wait()
        wait_scatter((NFA - 1) % 3)
        pltpu.sync_copy(rows_t, agg_sh.at[dstv_t], add=True)

        plsc.subcore_barrier()
        _write_out(sid, cid, agg_sh, agg_out)

    return functools.partial(
        pl.kernel, mesh=_MESH,
        out_type=(jax.ShapeDtypeStruct((NC, N, D), jnp.bfloat16),),
        scratch_types=scratch, compiler_params=_SC_PARAMS)(body)


def _make_sdeg():
    scratch = [
        [pltpu.VMEM((K,), jnp.int32)] * 3,
        [pltpu.VMEM((K, DE), jnp.float32)] * 3,
        pltpu.VMEM((K, DE), jnp.float32),
        pltpu.VMEM((TE,), jnp.int32),
        pltpu.VMEM((TE, DE), jnp.float32),
        pltpu.VMEM((TE, DE), jnp.float32),
        pltpu.VMEM((ZR, DE), jnp.float32),
        pltpu.VMEM_SHARED((N, DE), jnp.float32),
        pltpu.VMEM_SHARED((N, DE), jnp.float32),
        [pltpu.SemaphoreType.DMA] * 3,
        [pltpu.SemaphoreType.DMA] * 3,
    ]

    def body(ei_hbm, ea_hbm, s_out, deg_out, dstv, eav, ones, dstv_t, eav_t,
             ones_t, zbuf16, s_sh, deg_sh, sem_i, sem_s):
        cid = lax.axis_index("c")
        sid = lax.axis_index("s")
        ov = jnp.ones((16,), jnp.float32)

        @pl.loop(0, K)
        def _(i):
            ones[i, pl.ds(0, 16)] = ov

        @pl.loop(0, TE)
        def _(i):
            ones_t[i, pl.ds(0, 16)] = ov

        _zero_fill(zbuf16, DE)
        _zero_shared(sid, zbuf16, s_sh)
        _zero_shared(sid, zbuf16, deg_sh)
        plsc.subcore_barrier()

        ebase = (sid * NC + cid) * EPW

        def start_inputs(c, b):
            off = ebase + c * K
            pltpu.async_copy(ei_hbm.at[pl.ds(off, K)], dstv[b], sem_i[b])
            pltpu.async_copy(ea_hbm.at[pl.ds(off, K)], eav[b], sem_i[b])

        def wait_inputs(b):
            pltpu.make_async_copy(ei_hbm.at[pl.ds(0, K)], dstv[b], sem_i[b]).wait()
            pltpu.make_async_copy(ea_hbm.at[pl.ds(0, K)], eav[b], sem_i[b]).wait()

        def start_scatter(b):
            pltpu.async_copy(eav[b], s_sh.at[dstv[b]], sem_s[b], add=True)
            pltpu.async_copy(ones, deg_sh.at[dstv[b]], sem_s[b], add=True)

        def wait_scatter(b):
            pltpu.make_async_copy(eav[b], s_sh.at[dstv[b]], sem_s[b]).wait()
            pltpu.make_async_copy(ones, deg_sh.at[dstv[b]], sem_s[b]).wait()

        start_inputs(0, 0)
        start_inputs(1, 1)

        @pl.loop(0, NFULL, step=3)
        def _(j):
            for b in (0, 1, 2):
                c = j + b
                bp = (b + 2) % 3
                wait_inputs(b)
                start_scatter(b)
                can_load = c + 2 < NFULL

                @pl.when(jnp.logical_and(c >= 1, can_load))
                def _():
                    wait_scatter(bp)
                    start_inputs(c + 2, bp)

                @pl.when(jnp.logical_and(c < 1, can_load))
                def _():
                    start_inputs(c + 2, bp)

                @pl.when(jnp.logical_and(c >= 1, jnp.logical_not(can_load)))
                def _():
                    wait_scatter(bp)

        toff = ebase + NFULL * K
        pltpu.sync_copy(ei_hbm.at[pl.ds(toff, TE)], dstv_t)
        pltpu.sync_copy(ea_hbm.at[pl.ds(toff, TE)], eav_t)
        wait_scatter((NFULL - 1) % 3)
        pltpu.sync_copy(eav_t, s_sh.at[dstv_t], add=True)
        pltpu.sync_copy(ones_t, deg_sh.at[dstv_t], add=True)

        plsc.subcore_barrier()
        _write_out(sid, cid, s_sh, s_out)
        _write_out(sid, cid, deg_sh, deg_out)

    return functools.partial(
        pl.kernel, mesh=_MESH,
        out_type=(jax.ShapeDtypeStruct((NC, N, DE), jnp.float32),
                  jax.ShapeDtypeStruct((NC, N, DE), jnp.float32)),
        scratch_types=scratch, compiler_params=_SC_PARAMS)(body)


_spmm = _make_spmm()
_sdeg = _make_sdeg()


def _make_mlp(final_relu: bool, with_sd_inputs: bool):
    R = 2000

    def body(agg_ref, h_ref, s_ref, d_ref, a_ref, b_ref, v_ref, u_ref,
             w2_ref, b2_ref, o_ref):
        z = (agg_ref[0].astype(jnp.float32) + agg_ref[1].astype(jnp.float32)
             + h_ref[...])
        sarr = s_ref[0] + s_ref[1]
        darr = d_ref[0] + d_ref[1]
        dcol = darr[:, :1]
        pre = (jnp.dot(z, a_ref[...], preferred_element_type=jnp.float32)
               + jnp.dot(sarr, b_ref[...], preferred_element_type=jnp.float32)
               + dcol * v_ref[...] + u_ref[...])
        t = jnp.maximum(pre, 0.0)
        out = jnp.dot(t, w2_ref[...], preferred_element_type=jnp.float32) + b2_ref[...]
        if final_relu:
            out = jnp.maximum(out, 0.0)
        o_ref[...] = out

    grid = (N // R,)
    in_specs = [
        pl.BlockSpec((NC, R, D), lambda i: (0, i, 0)),
        pl.BlockSpec((R, D), lambda i: (i, 0)),
        pl.BlockSpec((NC, R, DE), lambda i: (0, i, 0)),
        pl.BlockSpec((NC, R, DE), lambda i: (0, i, 0)),
        pl.BlockSpec((D, 2 * D), lambda i: (0, 0)),
        pl.BlockSpec((DE, 2 * D), lambda i: (0, 0)),
        pl.BlockSpec((1, 2 * D), lambda i: (0, 0)),
        pl.BlockSpec((1, 2 * D), lambda i: (0, 0)),
        pl.BlockSpec((2 * D, D), lambda i: (0, 0)),
        pl.BlockSpec((1, D), lambda i: (0, 0)),
    ]
    return pl.pallas_call(
        body,
        grid=grid,
        in_specs=in_specs,
        out_specs=pl.BlockSpec((R, D), lambda i: (i, 0)),
        out_shape=jax.ShapeDtypeStruct((N, D), jnp.float32),
    )


_mlp0 = _make_mlp(final_relu=True, with_sd_inputs=True)
_mlp1 = _make_mlp(final_relu=False, with_sd_inputs=True)

_SCALE = 1.0 / np.sqrt(1.0 + EPS)


def kernel(x, edge_index, edge_attr, self_loop_index, self_loop_type,
           W_enc0, b_enc0, W1_0, b1_0, gamma0, beta0, W2_0, b2_0,
           W_enc1, b_enc1, W1_1, b1_1, gamma1, beta1, W2_1, b2_1):
    sl_row = ((jnp.arange(DE) == self_loop_index).astype(jnp.float32)
              * jnp.asarray(self_loop_type, jnp.float32))

    def fold(W1, b1, gamma, beta):
        g = gamma * _SCALE
        return W1 * g[None, :], b1 * g + beta

    W1f0, b1f0 = fold(W1_0, b1_0, gamma0, beta0)
    A0 = W1f0
    B0 = W_enc0 @ W1f0
    v0 = (b_enc0 @ W1f0)[None, :]
    u0 = ((sl_row @ W_enc0 + b_enc0) @ W1f0 + b1f0)[None, :]

    W1f1, b1f1 = fold(W1_1, b1_1, gamma1, beta1)
    A1 = W1f1[:D]
    Wb = W1f1[D:]
    B1 = W_enc1 @ Wb
    v1 = (b_enc1 @ Wb)[None, :]
    u1 = ((sl_row @ W_enc1 + b_enc1) @ Wb + b1f1)[None, :]

    ei_lin = edge_index.reshape(2 * E)
    (aggx,) = _spmm(x.astype(jnp.bfloat16), ei_lin)
    ei_lin2, aggx = lax.optimization_barrier((ei_lin, aggx))
    S, deg = _sdeg(ei_lin2, edge_attr)
    h0 = _mlp0(aggx, x, S, deg, A0, B0, v0, u0, W2_0, b2_0[None, :])
    (aggh,) = _spmm(h0.astype(jnp.bfloat16), ei_lin)
    h1 = _mlp1(aggh, h0, S, deg, A1, B1, v1, u1, W2_1, b2_1[None, :])
    return h1


# speedup vs baseline: 1.3149x; 1.0009x over previous
import functools

import jax
import jax.numpy as jnp
import numpy as np
from jax import lax
from jax.experimental import pallas as pl
from jax.experimental.pallas import tpu as pltpu
from jax.experimental.pallas import tpu_sc as plsc

N = 10000
E = 320000
D = 128
DE = 16
EPS = 1e-05

NC = 2
NS = 16
NW = NC * NS
EPW = E // NW
K = 128
NFULL = EPW // K
TE = EPW - NFULL * K
KA = 128
NFA = EPW // KA
TEA = EPW - NFA * KA
RO = 624
ZR = 48
TAIL = N - NS * RO

_MESH = plsc.VectorSubcoreMesh(core_axis_name="c", subcore_axis_name="s")
_SC_PARAMS = pltpu.CompilerParams(use_tc_tiling_on_sc=False)


def _zero_fill(zbuf, width):
    zv = jnp.zeros((16,), jnp.float32)

    @pl.loop(0, ZR)
    def _(i):
        @pl.loop(0, width // 16)
        def _(j):
            zbuf[i, pl.ds(j * 16, 16)] = zv


def _zero_fill_bf16(zbuf, width):
    zv = jnp.zeros((32,), jnp.bfloat16)

    @pl.loop(0, ZR)
    def _(i):
        @pl.loop(0, width // 32)
        def _(j):
            zbuf[i, pl.ds(j * 32, 32)] = zv


def _zero_shared(sid, zbuf, sh):
    rbase = sid * RO
    for r in range(RO // ZR):
        pltpu.sync_copy(zbuf, sh.at[pl.ds(rbase + r * ZR, ZR)])

    @pl.when(sid == NS - 1)
    def _():
        pltpu.sync_copy(zbuf.at[pl.ds(0, TAIL)], sh.at[pl.ds(NS * RO, TAIL)])


def _write_out(sid, cid, sh, out):
    rbase = sid * RO
    pltpu.sync_copy(sh.at[pl.ds(rbase, RO)], out.at[cid, pl.ds(rbase, RO)])

    @pl.when(sid == NS - 1)
    def _():
        pltpu.sync_copy(sh.at[pl.ds(NS * RO, TAIL)], out.at[cid, pl.ds(NS * RO, TAIL)])


def _make_spmm():
    scratch = [
        [pltpu.VMEM((KA,), jnp.int32)] * 3,
        [pltpu.VMEM((KA,), jnp.int32)] * 3,
        [pltpu.VMEM((KA, D), jnp.bfloat16)] * 3,
        pltpu.VMEM((TEA,), jnp.int32),
        pltpu.VMEM((TEA,), jnp.int32),
        pltpu.VMEM((TEA, D), jnp.bfloat16),
        pltpu.VMEM((ZR, D), jnp.bfloat16),
        pltpu.VMEM_SHARED((N, D), jnp.bfloat16),
        [pltpu.SemaphoreType.DMA] * 3,
        [pltpu.SemaphoreType.DMA] * 3,
        [pltpu.SemaphoreType.DMA] * 3,
        pltpu.SemaphoreType.DMA,
    ]

    def body(h_hbm, ei_hbm, agg_out, dstv, srcv, rows, dstv_t, srcv_t, rows_t,
             zbuf, agg_sh, sem_i, sem_g, sem_s, sem_t):
        cid = lax.axis_index("c")
        sid = lax.axis_index("s")
        _zero_fill_bf16(zbuf, D)
        _zero_shared(sid, zbuf, agg_sh)
        plsc.subcore_barrier()

        ebase = (sid * NC + cid) * EPW

        def start_inputs(c, b):
            off = ebase + c * KA
            pltpu.async_copy(ei_hbm.at[pl.ds(off, KA)], dstv[b], sem_i[b])
            pltpu.async_copy(ei_hbm.at[pl.ds(E + off, KA)], srcv[b], sem_i[b])

        def wait_inputs(b):
            pltpu.make_async_copy(ei_hbm.at[pl.ds(0, KA)], dstv[b], sem_i[b]).wait()
            pltpu.make_async_copy(ei_hbm.at[pl.ds(0, KA)], srcv[b], sem_i[b]).wait()

        def start_gather(b):
            pltpu.async_copy(h_hbm.at[srcv[b]], rows[b], sem_g[b])

        def wait_gather(b):
            pltpu.make_async_copy(h_hbm.at[srcv[b]], rows[b], sem_g[b]).wait()

        def start_scatter(b):
            pltpu.async_copy(rows[b], agg_sh.at[dstv[b]], sem_s[b], add=True)

        def wait_scatter(b):
            pltpu.make_async_copy(rows[b], agg_sh.at[dstv[b]], sem_s[b]).wait()

        start_inputs(0, 0)
        start_inputs(1, 1)
        start_inputs(2, 2)
        wait_inputs(0)
        start_gather(0)

        @pl.loop(0, NFA - 3, step=3)
        def _(j):
            for b in (0, 1, 2):
                y = j + b
                bn, bp = (b + 1) % 3, (b + 2) % 3
                wait_inputs(bn)
                if b == 0:
                    @pl.when(y >= 1)
                    def _():
                        wait_scatter(bp)
                        start_inputs(y + 2, bp)
                else:
                    wait_scatter(bp)
                    start_inputs(y + 2, bp)
                start_gather(bn)
                wait_gather(b)
                start_scatter(b)

        y = NFA - 3
        b, bn, bp = y % 3, (y + 1) % 3, (y + 2) % 3
        wait_inputs(bn)
        wait_scatter(bp)
        start_inputs(y + 2, bp)
        start_gather(bn)
        wait_gather(b)
        start_scatter(b)

        y = NFA - 2
        b, bn, bp = y % 3, (y + 1) % 3, (y + 2) % 3
        wait_inputs(bn)
        wait_scatter(bp)
        start_gather(bn)
        wait_gather(b)
        start_scatter(b)

        y = NFA - 1
        b, bn, bp = y % 3, (y + 1) % 3, (y + 2) % 3
        wait_scatter(bp)
        wait_gather(b)
        start_scatter(b)

        toff = ebase + NFA * KA
        pltpu.sync_copy(ei_hbm.at[pl.ds(toff, TEA)], dstv_t)
        pltpu.sync_copy(ei_hbm.at[pl.ds(E + toff, TEA)], srcv_t)
        pltpu.async_copy(h_hbm.at[srcv_t], rows_t, sem_t).wait()
        wait_scatter((NFA - 1) % 3)
        pltpu.sync_copy(rows_t, agg_sh.at[dstv_t], add=True)

        plsc.subcore_barrier()
        _write_out(sid, cid, agg_sh, agg_out)

    return functools.partial(
        pl.kernel, mesh=_MESH,
        out_type=(jax.ShapeDtypeStruct((NC, N, D), jnp.bfloat16),),
        scratch_types=scratch, compiler_params=_SC_PARAMS)(body)


def _make_sdeg():
    scratch = [
        [pltpu.VMEM((K,), jnp.int32)] * 3,
        [pltpu.VMEM((K, DE), jnp.float32)] * 3,
        pltpu.VMEM((K, DE), jnp.float32),
        pltpu.VMEM((TE,), jnp.int32),
        pltpu.VMEM((TE, DE), jnp.float32),
        pltpu.VMEM((TE, DE), jnp.float32),
        pltpu.VMEM((ZR, DE), jnp.float32),
        pltpu.VMEM_SHARED((N, DE), jnp.float32),
        pltpu.VMEM_SHARED((N, DE), jnp.float32),
        [pltpu.SemaphoreType.DMA] * 3,
        [pltpu.SemaphoreType.DMA] * 3,
    ]

    def body(ei_hbm, ea_hbm, sd_out, dstv, eav, ones, dstv_t, eav_t,
             ones_t, zbuf16, s_sh, deg_sh, sem_i, sem_s):
        cid = lax.axis_index("c")
        sid = lax.axis_index("s")
        ov = jnp.ones((16,), jnp.float32)

        @pl.loop(0, K)
        def _(i):
            ones[i, pl.ds(0, 16)] = ov

        @pl.loop(0, TE)
        def _(i):
            ones_t[i, pl.ds(0, 16)] = ov

        _zero_fill(zbuf16, DE)
        _zero_shared(sid, zbuf16, s_sh)
        _zero_shared(sid, zbuf16, deg_sh)
        plsc.subcore_barrier()

        ebase = (sid * NC + cid) * EPW

        def start_inputs(c, b):
            off = ebase + c * K
            pltpu.async_copy(ei_hbm.at[pl.ds(off, K)], dstv[b], sem_i[b])
            pltpu.async_copy(ea_hbm.at[pl.ds(off, K)], eav[b], sem_i[b])

        def wait_inputs(b):
            pltpu.make_async_copy(ei_hbm.at[pl.ds(0, K)], dstv[b], sem_i[b]).wait()
            pltpu.make_async_copy(ea_hbm.at[pl.ds(0, K)], eav[b], sem_i[b]).wait()

        def start_scatter(b):
            pltpu.async_copy(eav[b], s_sh.at[dstv[b]], sem_s[b], add=True)
            pltpu.async_copy(ones, deg_sh.at[dstv[b]], sem_s[b], add=True)

        def wait_scatter(b):
            pltpu.make_async_copy(eav[b], s_sh.at[dstv[b]], sem_s[b]).wait()
            pltpu.make_async_copy(ones, deg_sh.at[dstv[b]], sem_s[b]).wait()

        start_inputs(0, 0)
        start_inputs(1, 1)

        @pl.loop(0, NFULL, step=3)
        def _(j):
            for b in (0, 1, 2):
                c = j + b
                bp = (b + 2) % 3
                wait_inputs(b)
                start_scatter(b)
                can_load = c + 2 < NFULL

                @pl.when(jnp.logical_and(c >= 1, can_load))
                def _():
                    wait_scatter(bp)
                    start_inputs(c + 2, bp)

                @pl.when(jnp.logical_and(c < 1, can_load))
                def _():
                    start_inputs(c + 2, bp)

                @pl.when(jnp.logical_and(c >= 1, jnp.logical_not(can_load)))
                def _():
                    wait_scatter(bp)

        toff = ebase + NFULL * K
        pltpu.sync_copy(ei_hbm.at[pl.ds(toff, TE)], dstv_t)
        pltpu.sync_copy(ea_hbm.at[pl.ds(toff, TE)], eav_t)
        wait_scatter((NFULL - 1) % 3)
        pltpu.sync_copy(eav_t, s_sh.at[dstv_t], add=True)
        pltpu.sync_copy(ones_t, deg_sh.at[dstv_t], add=True)

        plsc.subcore_barrier()
        rbase = sid * RO
        pltpu.sync_copy(s_sh.at[pl.ds(rbase, RO)],
                        sd_out.at[cid, 0, pl.ds(rbase, RO)])
        pltpu.sync_copy(deg_sh.at[pl.ds(rbase, RO)],
                        sd_out.at[cid, 1, pl.ds(rbase, RO)])

        @pl.when(sid == NS - 1)
        def _():
            pltpu.sync_copy(s_sh.at[pl.ds(NS * RO, TAIL)],
                            sd_out.at[cid, 0, pl.ds(NS * RO, TAIL)])
            pltpu.sync_copy(deg_sh.at[pl.ds(NS * RO, TAIL)],
                            sd_out.at[cid, 1, pl.ds(NS * RO, TAIL)])

    return functools.partial(
        pl.kernel, mesh=_MESH,
        out_type=(jax.ShapeDtypeStruct((NC, 2, N, DE), jnp.float32),),
        scratch_types=scratch, compiler_params=_SC_PARAMS)(body)


_spmm = _make_spmm()
_sdeg = _make_sdeg()


def _make_mlp(final_relu: bool, with_sd_inputs: bool):
    R = 2000

    def body(agg_ref, h_ref, sd_ref, a_ref, b_ref, v_ref, u_ref,
             w2_ref, b2_ref, o_ref):
        z = (agg_ref[0].astype(jnp.float32) + agg_ref[1].astype(jnp.float32)
             + h_ref[...])
        sarr = sd_ref[0, 0] + sd_ref[1, 0]
        darr = sd_ref[0, 1] + sd_ref[1, 1]
        dcol = darr[:, :1]
        pre = (jnp.dot(z, a_ref[...], preferred_element_type=jnp.float32)
               + jnp.dot(sarr, b_ref[...], preferred_element_type=jnp.float32)
               + dcol * v_ref[...] + u_ref[...])
        t = jnp.maximum(pre, 0.0)
        out = jnp.dot(t, w2_ref[...], preferred_element_type=jnp.float32) + b2_ref[...]
        if final_relu:
            out = jnp.maximum(out, 0.0)
        o_ref[...] = out

    grid = (N // R,)
    in_specs = [
        pl.BlockSpec((NC, R, D), lambda i: (0, i, 0)),
        pl.BlockSpec((R, D), lambda i: (i, 0)),
        pl.BlockSpec((NC, 2, R, DE), lambda i: (0, 0, i, 0)),
        pl.BlockSpec((D, 2 * D), lambda i: (0, 0)),
        pl.BlockSpec((DE, 2 * D), lambda i: (0, 0)),
        pl.BlockSpec((1, 2 * D), lambda i: (0, 0)),
        pl.BlockSpec((1, 2 * D), lambda i: (0, 0)),
        pl.BlockSpec((2 * D, D), lambda i: (0, 0)),
        pl.BlockSpec((1, D), lambda i: (0, 0)),
    ]
    return pl.pallas_call(
        body,
        grid=grid,
        in_specs=in_specs,
        out_specs=pl.BlockSpec((R, D), lambda i: (i, 0)),
        out_shape=jax.ShapeDtypeStruct((N, D), jnp.float32),
    )


_mlp0 = _make_mlp(final_relu=True, with_sd_inputs=True)
_mlp1 = _make_mlp(final_relu=False, with_sd_inputs=True)

_SCALE = 1.0 / np.sqrt(1.0 + EPS)


def kernel(x, edge_index, edge_attr, self_loop_index, self_loop_type,
           W_enc0, b_enc0, W1_0, b1_0, gamma0, beta0, W2_0, b2_0,
           W_enc1, b_enc1, W1_1, b1_1, gamma1, beta1, W2_1, b2_1):
    sl_row = ((jnp.arange(DE) == self_loop_index).astype(jnp.float32)
              * jnp.asarray(self_loop_type, jnp.float32))

    def fold(W1, b1, gamma, beta):
        g = gamma * _SCALE
        return W1 * g[None, :], b1 * g + beta

    W1f0, b1f0 = fold(W1_0, b1_0, gamma0, beta0)
    A0 = W1f0
    B0 = W_enc0 @ W1f0
    v0 = (b_enc0 @ W1f0)[None, :]
    u0 = ((sl_row @ W_enc0 + b_enc0) @ W1f0 + b1f0)[None, :]

    W1f1, b1f1 = fold(W1_1, b1_1, gamma1, beta1)
    A1 = W1f1[:D]
    Wb = W1f1[D:]
    B1 = W_enc1 @ Wb
    v1 = (b_enc1 @ Wb)[None, :]
    u1 = ((sl_row @ W_enc1 + b_enc1) @ Wb + b1f1)[None, :]

    ei_lin = edge_index.reshape(2 * E)
    (aggx,) = _spmm(x.astype(jnp.bfloat16), ei_lin)
    ei_lin2, aggx = lax.optimization_barrier((ei_lin, aggx))
    (SD,) = _sdeg(ei_lin2, edge_attr)
    h0 = _mlp0(aggx, x, SD, A0, B0, v0, u0, W2_0, b2_0[None, :])
    (aggh,) = _spmm(h0.astype(jnp.bfloat16), ei_lin)
    h1 = _mlp1(aggh, h0, SD, A1, B1, v1, u1, W2_1, b2_1[None, :])
    return h1


# speedup vs baseline: 1.3166x; 1.0013x over previous
import functools

import jax
import jax.numpy as jnp
import numpy as np
from jax import lax
from jax.experimental import pallas as pl
from jax.experimental.pallas import tpu as pltpu
from jax.experimental.pallas import tpu_sc as plsc

N = 10000
E = 320000
D = 128
DE = 16
EPS = 1e-05

NC = 2
NS = 16
NW = NC * NS
EPW = E // NW
K = 128
NFULL = EPW // K
TE = EPW - NFULL * K
KA = 128
NFA = EPW // KA
TEA = EPW - NFA * KA
RO = 624
ZR = 48
TAIL = N - NS * RO

_MESH = plsc.VectorSubcoreMesh(core_axis_name="c", subcore_axis_name="s")
_SC_PARAMS = pltpu.CompilerParams(use_tc_tiling_on_sc=False)


def _zero_fill(zbuf, width):
    zv = jnp.zeros((16,), jnp.float32)

    @pl.loop(0, ZR)
    def _(i):
        @pl.loop(0, width // 16)
        def _(j):
            zbuf[i, pl.ds(j * 16, 16)] = zv


def _zero_fill_bf16(zbuf, width):
    zv = jnp.zeros((32,), jnp.bfloat16)

    @pl.loop(0, ZR)
    def _(i):
        @pl.loop(0, width // 32)
        def _(j):
            zbuf[i, pl.ds(j * 32, 32)] = zv


def _zero_shared(sid, zbuf, sh):
    rbase = sid * RO
    for r in range(RO // ZR):
        pltpu.sync_copy(zbuf, sh.at[pl.ds(rbase + r * ZR, ZR)])

    @pl.when(sid == NS - 1)
    def _():
        pltpu.sync_copy(zbuf.at[pl.ds(0, TAIL)], sh.at[pl.ds(NS * RO, TAIL)])


def _write_out(sid, cid, sh, out):
    rbase = sid * RO
    pltpu.sync_copy(sh.at[pl.ds(rbase, RO)], out.at[cid, pl.ds(rbase, RO)])

    @pl.when(sid == NS - 1)
    def _():
        pltpu.sync_copy(sh.at[pl.ds(NS * RO, TAIL)], out.at[cid, pl.ds(NS * RO, TAIL)])


def _make_spmm():
    scratch = [
        [pltpu.VMEM((KA,), jnp.int32)] * 3,
        [pltpu.VMEM((KA,), jnp.int32)] * 3,
        [pltpu.VMEM((KA, D), jnp.bfloat16)] * 3,
        pltpu.VMEM((TEA,), jnp.int32),
        pltpu.VMEM((TEA,), jnp.int32),
        pltpu.VMEM((TEA, D), jnp.bfloat16),
        pltpu.VMEM((ZR, D), jnp.bfloat16),
        pltpu.VMEM_SHARED((N, D), jnp.bfloat16),
        [pltpu.SemaphoreType.DMA] * 3,
        [pltpu.SemaphoreType.DMA] * 3,
        [pltpu.SemaphoreType.DMA] * 3,
        pltpu.SemaphoreType.DMA,
    ]

    def body(h_hbm, ei_hbm, agg_out, dstv, srcv, rows, dstv_t, srcv_t, rows_t,
             zbuf, agg_sh, sem_i, sem_g, sem_s, sem_t):
        cid = lax.axis_index("c")
        sid = lax.axis_index("s")
        _zero_fill_bf16(zbuf, D)
        _zero_shared(sid, zbuf, agg_sh)
        plsc.subcore_barrier()

        ebase = (sid * NC + cid) * EPW

        def start_inputs(c, b):
            off = ebase + c * KA
            pltpu.async_copy(ei_hbm.at[pl.ds(off, KA)], dstv[b], sem_i[b])
            pltpu.async_copy(ei_hbm.at[pl.ds(E + off, KA)], srcv[b], sem_i[b])

        def wait_inputs(b):
            pltpu.make_async_copy(ei_hbm.at[pl.ds(0, KA)], dstv[b], sem_i[b]).wait()
            pltpu.make_async_copy(ei_hbm.at[pl.ds(0, KA)], srcv[b], sem_i[b]).wait()

        def start_gather(b):
            pltpu.async_copy(h_hbm.at[srcv[b]], rows[b], sem_g[b])

        def wait_gather(b):
            pltpu.make_async_copy(h_hbm.at[srcv[b]], rows[b], sem_g[b]).wait()

        def start_scatter(b):
            pltpu.async_copy(rows[b], agg_sh.at[dstv[b]], sem_s[b], add=True)

        def wait_scatter(b):
            pltpu.make_async_copy(rows[b], agg_sh.at[dstv[b]], sem_s[b]).wait()

        start_inputs(0, 0)
        start_inputs(1, 1)
        start_inputs(2, 2)
        wait_inputs(0)
        start_gather(0)

        @pl.loop(0, NFA - 3, step=3)
        def _(j):
            for b in (0, 1, 2):
                y = j + b
                bn, bp = (b + 1) % 3, (b + 2) % 3
                wait_inputs(bn)
                if b == 0:
                    @pl.when(y >= 1)
                    def _():
                        wait_scatter(bp)
                        start_inputs(y + 2, bp)
                else:
                    wait_scatter(bp)
                    start_inputs(y + 2, bp)
                start_gather(bn)
                wait_gather(b)
                start_scatter(b)

        y = NFA - 3
        b, bn, bp = y % 3, (y + 1) % 3, (y + 2) % 3
        wait_inputs(bn)
        wait_scatter(bp)
        start_inputs(y + 2, bp)
        start_gather(bn)
        wait_gather(b)
        start_scatter(b)

        y = NFA - 2
        b, bn, bp = y % 3, (y + 1) % 3, (y + 2) % 3
        wait_inputs(bn)
        wait_scatter(bp)
        start_gather(bn)
        wait_gather(b)
        start_scatter(b)

        y = NFA - 1
        b, bn, bp = y % 3, (y + 1) % 3, (y + 2) % 3
        wait_scatter(bp)
        wait_gather(b)
        start_scatter(b)

        toff = ebase + NFA * KA
        pltpu.sync_copy(ei_hbm.at[pl.ds(toff, TEA)], dstv_t)
        pltpu.sync_copy(ei_hbm.at[pl.ds(E + toff, TEA)], srcv_t)
        pltpu.async_copy(h_hbm.at[srcv_t], rows_t, sem_t).wait()
        wait_scatter((NFA - 1) % 3)
        pltpu.sync_copy(rows_t, agg_sh.at[dstv_t], add=True)

        plsc.subcore_barrier()
        _write_out(sid, cid, agg_sh, agg_out)

    return functools.partial(
        pl.kernel, mesh=_MESH,
        out_type=(jax.ShapeDtypeStruct((NC, N, D), jnp.bfloat16),),
        scratch_types=scratch, compiler_params=_SC_PARAMS)(body)


def _make_sdeg():
    scratch = [
        [pltpu.VMEM((K,), jnp.int32)] * 3,
        [pltpu.VMEM((K, DE), jnp.float32)] * 3,
        pltpu.VMEM((K, DE), jnp.float32),
        pltpu.VMEM((TE,), jnp.int32),
        pltpu.VMEM((TE, DE), jnp.float32),
        pltpu.VMEM((TE, DE), jnp.float32),
        pltpu.VMEM((ZR, DE), jnp.float32),
        pltpu.VMEM_SHARED((N, DE), jnp.float32),
        pltpu.VMEM_SHARED((N, DE), jnp.float32),
        [pltpu.SemaphoreType.DMA] * 3,
        [pltpu.SemaphoreType.DMA] * 3,
    ]

    def body(ei_hbm, ea_hbm, sd_out, dstv, eav, ones, dstv_t, eav_t,
             ones_t, zbuf16, s_sh, deg_sh, sem_i, sem_s):
        cid = lax.axis_index("c")
        sid = lax.axis_index("s")
        ov = jnp.ones((16,), jnp.float32)

        @pl.loop(0, K)
        def _(i):
            ones[i, pl.ds(0, 16)] = ov

        @pl.loop(0, TE)
        def _(i):
            ones_t[i, pl.ds(0, 16)] = ov

        _zero_fill(zbuf16, DE)
        _zero_shared(sid, zbuf16, s_sh)
        _zero_shared(sid, zbuf16, deg_sh)
        plsc.subcore_barrier()

        ebase = (sid * NC + cid) * EPW

        def start_inputs(c, b):
            off = ebase + c * K
            pltpu.async_copy(ei_hbm.at[pl.ds(off, K)], dstv[b], sem_i[b])
            pltpu.async_copy(ea_hbm.at[pl.ds(off, K)], eav[b], sem_i[b])

        def wait_inputs(b):
            pltpu.make_async_copy(ei_hbm.at[pl.ds(0, K)], dstv[b], sem_i[b]).wait()
            pltpu.make_async_copy(ea_hbm.at[pl.ds(0, K)], eav[b], sem_i[b]).wait()

        def start_scatter(b):
            pltpu.async_copy(eav[b], s_sh.at[dstv[b]], sem_s[b], add=True)
            pltpu.async_copy(ones, deg_sh.at[dstv[b]], sem_s[b], add=True)

        def wait_scatter(b):
            pltpu.make_async_copy(eav[b], s_sh.at[dstv[b]], sem_s[b]).wait()
            pltpu.make_async_copy(ones, deg_sh.at[dstv[b]], sem_s[b]).wait()

        start_inputs(0, 0)
        start_inputs(1, 1)

        @pl.loop(0, NFULL, step=3)
        def _(j):
            for b in (0, 1, 2):
                c = j + b
                bp = (b + 2) % 3
                wait_inputs(b)
                start_scatter(b)
                can_load = c + 2 < NFULL

                @pl.when(jnp.logical_and(c >= 1, can_load))
                def _():
                    wait_scatter(bp)
                    start_inputs(c + 2, bp)

                @pl.when(jnp.logical_and(c < 1, can_load))
                def _():
                    start_inputs(c + 2, bp)

                @pl.when(jnp.logical_and(c >= 1, jnp.logical_not(can_load)))
                def _():
                    wait_scatter(bp)

        toff = ebase + NFULL * K
        pltpu.sync_copy(ei_hbm.at[pl.ds(toff, TE)], dstv_t)
        pltpu.sync_copy(ea_hbm.at[pl.ds(toff, TE)], eav_t)
        wait_scatter((NFULL - 1) % 3)
        pltpu.sync_copy(eav_t, s_sh.at[dstv_t], add=True)
        pltpu.sync_copy(ones_t, deg_sh.at[dstv_t], add=True)

        plsc.subcore_barrier()
        rbase = sid * RO
        pltpu.sync_copy(s_sh.at[pl.ds(rbase, RO)],
                        sd_out.at[cid, 0, pl.ds(rbase, RO)])
        pltpu.sync_copy(deg_sh.at[pl.ds(rbase, RO)],
                        sd_out.at[cid, 1, pl.ds(rbase, RO)])

        @pl.when(sid == NS - 1)
        def _():
            pltpu.sync_copy(s_sh.at[pl.ds(NS * RO, TAIL)],
                            sd_out.at[cid, 0, pl.ds(NS * RO, TAIL)])
            pltpu.sync_copy(deg_sh.at[pl.ds(NS * RO, TAIL)],
                            sd_out.at[cid, 1, pl.ds(NS * RO, TAIL)])

    return functools.partial(
        pl.kernel, mesh=_MESH,
        out_type=(jax.ShapeDtypeStruct((NC, 2, N, DE), jnp.float32),),
        scratch_types=scratch, compiler_params=_SC_PARAMS)(body)


_spmm = _make_spmm()
_sdeg = _make_sdeg()


def _make_mlp(final_relu: bool, with_sd_inputs: bool):
    R = 2000

    def body(agg_ref, h_ref, sd_ref, a_ref, b_ref, v_ref, u_ref,
             w2_ref, b2_ref, o_ref):
        z = (agg_ref[0].astype(jnp.float32) + agg_ref[1].astype(jnp.float32)
             + h_ref[...])
        sarr = sd_ref[0, 0] + sd_ref[1, 0]
        darr = sd_ref[0, 1] + sd_ref[1, 1]
        dcol = darr[:, :1]
        pre = (jnp.dot(z.astype(jnp.bfloat16), a_ref[...].astype(jnp.bfloat16),
                       preferred_element_type=jnp.float32)
               + jnp.dot(sarr, b_ref[...], preferred_element_type=jnp.float32)
               + dcol * v_ref[...] + u_ref[...])
        t = jnp.maximum(pre, 0.0)
        out = jnp.dot(t.astype(jnp.bfloat16), w2_ref[...].astype(jnp.bfloat16),
                      preferred_element_type=jnp.float32) + b2_ref[...]
        if final_relu:
            out = jnp.maximum(out, 0.0)
        o_ref[...] = out

    grid = (N // R,)
    in_specs = [
        pl.BlockSpec((NC, R, D), lambda i: (0, i, 0)),
        pl.BlockSpec((R, D), lambda i: (i, 0)),
        pl.BlockSpec((NC, 2, R, DE), lambda i: (0, 0, i, 0)),
        pl.BlockSpec((D, 2 * D), lambda i: (0, 0)),
        pl.BlockSpec((DE, 2 * D), lambda i: (0, 0)),
        pl.BlockSpec((1, 2 * D), lambda i: (0, 0)),
        pl.BlockSpec((1, 2 * D), lambda i: (0, 0)),
        pl.BlockSpec((2 * D, D), lambda i: (0, 0)),
        pl.BlockSpec((1, D), lambda i: (0, 0)),
    ]
    return pl.pallas_call(
        body,
        grid=grid,
        in_specs=in_specs,
        out_specs=pl.BlockSpec((R, D), lambda i: (i, 0)),
        out_shape=jax.ShapeDtypeStruct((N, D), jnp.float32),
    )


_mlp0 = _make_mlp(final_relu=True, with_sd_inputs=True)
_mlp1 = _make_mlp(final_relu=False, with_sd_inputs=True)

_SCALE = 1.0 / np.sqrt(1.0 + EPS)


def kernel(x, edge_index, edge_attr, self_loop_index, self_loop_type,
           W_enc0, b_enc0, W1_0, b1_0, gamma0, beta0, W2_0, b2_0,
           W_enc1, b_enc1, W1_1, b1_1, gamma1, beta1, W2_1, b2_1):
    sl_row = ((jnp.arange(DE) == self_loop_index).astype(jnp.float32)
              * jnp.asarray(self_loop_type, jnp.float32))

    def fold(W1, b1, gamma, beta):
        g = gamma * _SCALE
        return W1 * g[None, :], b1 * g + beta

    W1f0, b1f0 = fold(W1_0, b1_0, gamma0, beta0)
    A0 = W1f0
    B0 = W_enc0 @ W1f0
    v0 = (b_enc0 @ W1f0)[None, :]
    u0 = ((sl_row @ W_enc0 + b_enc0) @ W1f0 + b1f0)[None, :]

    W1f1, b1f1 = fold(W1_1, b1_1, gamma1, beta1)
    A1 = W1f1[:D]
    Wb = W1f1[D:]
    B1 = W_enc1 @ Wb
    v1 = (b_enc1 @ Wb)[None, :]
    u1 = ((sl_row @ W_enc1 + b_enc1) @ Wb + b1f1)[None, :]

    ei_lin = edge_index.reshape(2 * E)
    (aggx,) = _spmm(x.astype(jnp.bfloat16), ei_lin)
    ei_lin2, aggx = lax.optimization_barrier((ei_lin, aggx))
    (SD,) = _sdeg(ei_lin2, edge_attr)
    h0 = _mlp0(aggx, x, SD, A0, B0, v0, u0, W2_0, b2_0[None, :])
    (aggh,) = _spmm(h0.astype(jnp.bfloat16), ei_lin)
    h1 = _mlp1(aggh, h0, SD, A1, B1, v1, u1, W2_1, b2_1[None, :])
    return h1
